# Initial kernel scaffold; baseline (speedup 1.0000x reference)
import jax
import jax.numpy as jnp
from jax.experimental import pallas as pl


def kernel(x, coords_ca, bb_embs, edge_index, batch_idx, W_emb, b_emb, mlp0_W1, mlp0_b1, mlp0_W2, mlp0_b2, mlp1_W1, mlp1_b1, mlp1_W2, mlp1_b2, ws0, wv0, ws1, wv1, lin1_W, lin1_b, out_W, out_b):
    raise NotImplementedError("write your pallas kernel here")


# speedup vs baseline: 3.2342x; 3.2342x over previous
import functools

import numpy as np
import jax
import jax.numpy as jnp
from jax import lax
from jax.experimental import pallas as pl
from jax.experimental.pallas import tpu as pltpu
from jax.experimental.pallas import tpu_sc as plsc

N_NODES = 50000
N_EDGES = 800000
E_PAD = 802816
EMB = 16
R_MAX = 10.0
P_CUT = 5.0
N_GRAPHS = 64

BE = 2048
EGRID = E_PAD // BE
BN = 2000
NGRID = N_NODES // BN

_f32 = jnp.float32
_bf16 = jnp.bfloat16

_SC_MESH = dict(core_axis_name="c", subcore_axis_name="s",
                num_cores=2, num_subcores=16)

_LIN1_PERM = np.concatenate([
    np.arange(16),
    np.array([16 + 3 * c + p for p in range(3) for c in range(16)]),
])


def _sc_gather(table, idx, chunk):
    B = idx.shape[0]
    D = table.shape[1]
    per_w = B // 32
    iters = per_w // chunk
    nsub = chunk // 128
    mesh = plsc.VectorSubcoreMesh(**_SC_MESH)

    @functools.partial(
        pl.kernel,
        out_type=jax.ShapeDtypeStruct((B, D), _f32),
        mesh=mesh,
        scratch_types=[
            pltpu.VMEM((chunk,), jnp.int32),
            pltpu.VMEM((chunk, D), _f32),
            pltpu.SemaphoreType.DMA,
        ],
        compiler_params=pltpu.CompilerParams(use_tc_tiling_on_sc=False),
    )
    def gk(table_hbm, idx_hbm, out_hbm, idx_v, rows_v, sem):
        wid = lax.axis_index("s") * 2 + lax.axis_index("c")
        base = wid * per_w

        @pl.loop(0, iters)
        def _(i):
            b = base + i * chunk
            pltpu.sync_copy(idx_hbm.at[pl.ds(b, chunk)], idx_v)
            descs = [
                pltpu.async_copy(
                    table_hbm.at[idx_v.at[pl.ds(j * 128, 128)]],
                    rows_v.at[pl.ds(j * 128, 128)], sem)
                for j in range(nsub)
            ]
            for d in descs:
                d.wait()
            pltpu.sync_copy(rows_v, out_hbm.at[pl.ds(b, chunk)])

    return gk(table, idx)


def _sc_scatter_add(m, dst2d):
    rows_chunk = 1024
    groups = E_PAD // rows_chunk
    per_sub = groups // 16
    rows_per_sub = N_NODES // 16
    zrows = 125
    mesh = plsc.VectorSubcoreMesh(**_SC_MESH)

    @functools.partial(
        pl.kernel,
        out_type=jax.ShapeDtypeStruct((4, N_NODES, EMB), _f32),
        mesh=mesh,
        scratch_types=[
            pltpu.VMEM((8, 128), jnp.int32),
            pltpu.VMEM((rows_chunk, EMB), _f32),
            pltpu.VMEM((zrows, EMB), _f32),
            pltpu.VMEM_SHARED((N_NODES, EMB), _f32),
            pltpu.SemaphoreType.DMA,
        ],
        compiler_params=pltpu.CompilerParams(use_tc_tiling_on_sc=False),
    )
    def sk(m_hbm, d_hbm, out_hbm, idx_v, m_v, zbuf, acc, sem):
        c = lax.axis_index("c")
        sid = lax.axis_index("s")
        zero16 = jnp.zeros((16,), _f32)
        for r in range(zrows):
            zbuf[r, pl.ds(0, 16)] = zero16
        base_r = sid * rows_per_sub

        for sub in range(2):
            grp_c = c * 2 + sub

            @pl.loop(0, rows_per_sub // zrows)
            def _(r):
                pltpu.sync_copy(zbuf, acc.at[pl.ds(base_r + r * zrows, zrows)])

            plsc.subcore_barrier()

            @pl.loop(0, per_sub)
            def _(g):
                grp = sid * per_sub + g
                pltpu.sync_copy(
                    m_hbm.at[grp_c, pl.ds(grp * rows_chunk, rows_chunk)], m_v)
                pltpu.sync_copy(d_hbm.at[pl.ds(grp * 8, 8)], idx_v)
                for j in range(8):
                    pltpu.sync_copy(m_v.at[pl.ds(j * 128, 128)],
                                    acc.at[idx_v.at[j]], add=True)

            plsc.subcore_barrier()
            pltpu.sync_copy(acc.at[pl.ds(base_r, rows_per_sub)],
                            out_hbm.at[grp_c, pl.ds(base_r, rows_per_sub)])

    return sk(m, dst2d)


def _bdot(a, w, b):
    return lax.dot_general(a, w, (((1,), (0,)), ((), ())),
                           preferred_element_type=_f32,
                           precision=lax.Precision.HIGHEST) + b[...]


def _embed_body(xf_ref, bb_ref, wemb_ref, bemb_ref, hs_ref):
    xf = xf_ref[...]
    aa = lax.broadcasted_iota(jnp.int32, (BN, 26), 1).astype(_f32)
    onehot = (aa == xf).astype(_f32)
    feat = jnp.concatenate([onehot, bb_ref[...]], axis=1)
    hs_ref[...] = _bdot(feat, wemb_ref[...], bemb_ref)


def _edge0_body(cs_ref, cd_ref, hs_ref, w1_ref, b1_ref, w2_ref, b2_ref,
                m_ref, ef_ref, y1_ref):
    i = pl.program_id(0)
    v = cs_ref[:, 0:4] - cd_ref[:, 0:4]
    len2 = jnp.sum(v * v, axis=1, keepdims=True)
    l = jnp.sqrt(len2)
    n = lax.broadcasted_iota(jnp.int32, (BE, 8), 1).astype(_f32) + 1.0
    arg = n * (jnp.pi / R_MAX) * l
    bes = jnp.sqrt(2.0 / R_MAX) * jnp.sin(arg) / l
    u = l * (1.0 / R_MAX)
    u2 = u * u
    u4 = u2 * u2
    u5 = u4 * u
    env = 1.0 - 21.0 * u5 + 35.0 * u5 * u - 15.0 * u5 * u2
    env = jnp.where(u < 1.0, env, 0.0)
    ef = bes * env
    ef_ref[...] = ef
    h1 = jnp.maximum(_bdot(ef, w1_ref[...], b1_ref), 0.0)
    w = _bdot(h1, w2_ref[...], b2_ref)
    linv = 1.0 / (l + 1e-12)
    y1 = jnp.sqrt(3.0) * v * linv
    y1_ref[...] = y1
    hs = hs_ref[...]
    eid = i * BE + lax.broadcasted_iota(jnp.int32, (BE, 1), 0)
    mask = eid < N_EDGES
    m_s = w[:, 0:16] * hs
    wh = w[:, 16:32] * hs
    zero = jnp.zeros_like(m_s)
    m_ref[0] = jnp.where(mask, m_s, zero)
    m_ref[1] = jnp.where(mask, wh * y1[:, 0:1], zero)
    m_ref[2] = jnp.where(mask, wh * y1[:, 1:2], zero)
    m_ref[3] = jnp.where(mask, wh * y1[:, 2:3], zero)


def _edge1_body(ef_ref, y1_ref, g_ref, w1_ref, b1_ref, w2_ref, b2_ref, m_ref):
    i = pl.program_id(0)
    ef = ef_ref[...]
    h1 = jnp.maximum(_bdot(ef, w1_ref[...], b1_ref), 0.0)
    w = _bdot(h1, w2_ref[...], b2_ref)
    g = g_ref[...]
    hs = g[:, 0:16]
    hvx = g[:, 16:32]
    hvy = g[:, 32:48]
    hvz = g[:, 48:64]
    y1 = y1_ref[...]
    y1x = y1[:, 0:1]
    y1y = y1[:, 1:2]
    y1z = y1[:, 2:3]
    dot = hvx * y1x + hvy * y1y + hvz * y1z
    m_s = w[:, 0:16] * hs + w[:, 48:64] * dot
    wh = w[:, 16:32] * hs
    m_vx = wh * y1x + w[:, 32:48] * hvx
    m_vy = wh * y1y + w[:, 32:48] * hvy
    m_vz = wh * y1z + w[:, 32:48] * hvz
    eid = i * BE + lax.broadcasted_iota(jnp.int32, (BE, 1), 0)
    mask = eid < N_EDGES
    zero = jnp.zeros_like(m_s)
    m_ref[0] = jnp.where(mask, m_s, zero)
    m_ref[1] = jnp.where(mask, m_vx, zero)
    m_ref[2] = jnp.where(mask, m_vy, zero)
    m_ref[3] = jnp.where(mask, m_vz, zero)


def _prod_body(with_v_res, agg_ref, res_ref, ws_ref, wv_ref, hall_ref):
    s = agg_ref[0]
    vx = agg_ref[1]
    vy = agg_ref[2]
    vz = agg_ref[3]
    vv = vx * vx + vy * vy + vz * vz
    s2 = s * s
    ws = ws_ref[...]
    wv = wv_ref[...]
    res = res_ref[...]
    out_s = (ws[0:1] * s + ws[1:2] * s2 + ws[2:3] * vv + ws[3:4] * (s2 * s)
             + ws[4:5] * (s * vv) + res[:, 0:16])
    coefv = wv[0:1] + wv[1:2] * s + wv[2:3] * vv + wv[3:4] * s2
    out_vx = coefv * vx
    out_vy = coefv * vy
    out_vz = coefv * vz
    if with_v_res:
        out_vx = out_vx + res[:, 16:32]
        out_vy = out_vy + res[:, 32:48]
        out_vz = out_vz + res[:, 48:64]
    hall_ref[...] = jnp.concatenate([out_s, out_vx, out_vy, out_vz], axis=1)


def _pool_body(hall_ref, bidx_ref, lw_ref, lb_ref, ow_ref, ob_ref,
               out_ref, acc_ref):
    i = pl.program_id(0)

    @pl.when(i == 0)
    def _():
        acc_ref[...] = jnp.zeros((N_GRAPHS, 4 * EMB), _f32)

    gids = lax.broadcasted_iota(jnp.int32, (BN, N_GRAPHS), 1).astype(_f32)
    onehot = (gids == bidx_ref[...]).astype(_bf16)
    hb = hall_ref[...].astype(_bf16)
    acc_ref[...] += lax.dot_general(onehot, hb, (((0,), (0,)), ((), ())),
                                    preferred_element_type=_f32)

    @pl.when(i == NGRID - 1)
    def _():
        pooled = acc_ref[...]
        h = jnp.maximum(_bdot(pooled, lw_ref[...], lb_ref), 0.0)
        out_ref[...] = _bdot(h, ow_ref[...], ob_ref)


def _full(shape):
    return pl.BlockSpec(shape, lambda i: tuple(0 for _ in shape))


def _tc_embed(xf, bb, wemb, bemb):
    return pl.pallas_call(
        _embed_body,
        grid=(NGRID,),
        in_specs=[
            pl.BlockSpec((BN, 1), lambda i: (i, 0)),
            pl.BlockSpec((BN, 6), lambda i: (i, 0)),
            _full((32, EMB)),
            _full((1, EMB)),
        ],
        out_specs=pl.BlockSpec((BN, EMB), lambda i: (i, 0)),
        out_shape=jax.ShapeDtypeStruct((N_NODES, EMB), _f32),
    )(xf, bb, wemb, bemb)


def _tc_edge0(cc, hs_src, w1, b1, w2, b2):
    return pl.pallas_call(
        _edge0_body,
        grid=(EGRID,),
        in_specs=[
            pl.BlockSpec((BE, 16), lambda i: (i, 0)),
            pl.BlockSpec((BE, 16), lambda i: (i + EGRID, 0)),
            pl.BlockSpec((BE, EMB), lambda i: (i, 0)),
            _full((8, 64)),
            _full((1, 64)),
            _full((64, 32)),
            _full((1, 32)),
        ],
        out_specs=[
            pl.BlockSpec((4, BE, EMB), lambda i: (0, i, 0)),
            pl.BlockSpec((BE, 8), lambda i: (i, 0)),
            pl.BlockSpec((BE, 4), lambda i: (i, 0)),
        ],
        out_shape=[
            jax.ShapeDtypeStruct((4, E_PAD, EMB), _f32),
            jax.ShapeDtypeStruct((E_PAD, 8), _f32),
            jax.ShapeDtypeStruct((E_PAD, 4), _f32),
        ],
    )(cc, cc, hs_src, w1, b1, w2, b2)


def _tc_edge1(ef, y1, g, w1, b1, w2, b2):
    return pl.pallas_call(
        _edge1_body,
        grid=(EGRID,),
        in_specs=[
            pl.BlockSpec((BE, 8), lambda i: (i, 0)),
            pl.BlockSpec((BE, 4), lambda i: (i, 0)),
            pl.BlockSpec((BE, 64), lambda i: (i, 0)),
            _full((8, 64)),
            _full((1, 64)),
            _full((64, 64)),
            _full((1, 64)),
        ],
        out_specs=pl.BlockSpec((4, BE, EMB), lambda i: (0, i, 0)),
        out_shape=jax.ShapeDtypeStruct((4, E_PAD, EMB), _f32),
    )(ef, y1, g, w1, b1, w2, b2)


def _tc_prod(agg, res, ws, wv, with_v_res):
    rdim = res.shape[1]
    return pl.pallas_call(
        functools.partial(_prod_body, with_v_res),
        grid=(NGRID,),
        in_specs=[
            pl.BlockSpec((4, BN, EMB), lambda i: (0, i, 0)),
            pl.BlockSpec((BN, rdim), lambda i: (i, 0)),
            _full((5, EMB)),
            _full((4, EMB)),
        ],
        out_specs=pl.BlockSpec((BN, 64), lambda i: (i, 0)),
        out_shape=jax.ShapeDtypeStruct((N_NODES, 64), _f32),
    )(agg, res, ws, wv)


def _tc_pool(hall, bidxf, lw, lb, ow, ob):
    nout = ow.shape[1]
    return pl.pallas_call(
        _pool_body,
        grid=(NGRID,),
        in_specs=[
            pl.BlockSpec((BN, 64), lambda i: (i, 0)),
            pl.BlockSpec((BN, 1), lambda i: (i, 0)),
            _full((64, 64)),
            _full((1, 64)),
            _full((64, nout)),
            _full((1, nout)),
        ],
        out_specs=pl.BlockSpec((N_GRAPHS, nout), lambda i: (0, 0)),
        out_shape=jax.ShapeDtypeStruct((N_GRAPHS, nout), _f32),
        scratch_shapes=[pltpu.VMEM((N_GRAPHS, 4 * EMB), _f32)],
    )(hall, bidxf, lw, lb, ow, ob)


def kernel(x, coords_ca, bb_embs, edge_index, batch_idx,
           W_emb, b_emb, mlp0_W1, mlp0_b1, mlp0_W2, mlp0_b2,
           mlp1_W1, mlp1_b1, mlp1_W2, mlp1_b2,
           ws0, wv0, ws1, wv1, lin1_W, lin1_b, out_W, out_b):
    pad = E_PAD - N_EDGES
    src = edge_index[0].astype(jnp.int32)
    dst = edge_index[1].astype(jnp.int32)
    zpad = jnp.zeros((pad,), jnp.int32)
    src_p = jnp.concatenate([src, zpad])
    dst_p = jnp.concatenate([dst, zpad])
    cidx = jnp.concatenate([src_p, dst_p])
    dst2d = dst_p.reshape(-1, 128)
    coords16 = jnp.pad(coords_ca, ((0, 0), (0, 13)))
    xf = x.astype(_f32).reshape(N_NODES, 1)
    bidxf = batch_idx.astype(_f32).reshape(N_NODES, 1)

    h_s = _tc_embed(xf, bb_embs, W_emb, b_emb.reshape(1, -1))
    cc = _sc_gather(coords16, cidx, 1024)
    hs_src = _sc_gather(h_s, src_p, 512)

    m0, ef, y1 = _tc_edge0(cc, hs_src,
                           mlp0_W1, mlp0_b1.reshape(1, -1),
                           mlp0_W2, mlp0_b2.reshape(1, -1))
    agg0 = _sc_scatter_add(m0, dst2d)
    hall1 = _tc_prod(agg0, h_s, ws0, wv0, with_v_res=False)

    g = _sc_gather(hall1, src_p, 512)
    m1 = _tc_edge1(ef, y1, g,
                   mlp1_W1, mlp1_b1.reshape(1, -1),
                   mlp1_W2, mlp1_b2.reshape(1, -1))
    agg1 = _sc_scatter_add(m1, dst2d)
    hall2 = _tc_prod(agg1, hall1, ws1, wv1, with_v_res=True)

    lin1_W = lin1_W[_LIN1_PERM]
    return _tc_pool(hall2, bidxf, lin1_W, lin1_b.reshape(1, -1),
                    out_W, out_b.reshape(1, -1))


# speedup vs baseline: 3.6363x; 1.1243x over previous
import functools

import numpy as np
import jax
import jax.numpy as jnp
from jax import lax
from jax.experimental import pallas as pl
from jax.experimental.pallas import tpu as pltpu
from jax.experimental.pallas import tpu_sc as plsc

N_NODES = 50000
N_EDGES = 800000
E_PAD = 802816
EMB = 16
R_MAX = 10.0
P_CUT = 5.0
N_GRAPHS = 64

BE = 2048
EGRID = E_PAD // BE
BN = 2000
NGRID = N_NODES // BN

_f32 = jnp.float32
_bf16 = jnp.bfloat16

_SC_MESH = dict(core_axis_name="c", subcore_axis_name="s",
                num_cores=2, num_subcores=16)

_LIN1_PERM = np.concatenate([
    np.arange(16),
    np.array([16 + 3 * c + p for p in range(3) for c in range(16)]),
])


def _sc_gather(table, idx, chunk):
    B = idx.shape[0]
    D = table.shape[1]
    per_w = B // 32
    iters = per_w // chunk
    nsub = chunk // 128
    mesh = plsc.VectorSubcoreMesh(**_SC_MESH)

    @functools.partial(
        pl.kernel,
        out_type=jax.ShapeDtypeStruct((B, D), _f32),
        mesh=mesh,
        scratch_types=[
            pltpu.VMEM((chunk,), jnp.int32),
            pltpu.VMEM((chunk, D), _f32),
            pltpu.SemaphoreType.DMA,
        ],
        compiler_params=pltpu.CompilerParams(use_tc_tiling_on_sc=False),
    )
    def gk(table_hbm, idx_hbm, out_hbm, idx_v, rows_v, sem):
        wid = lax.axis_index("s") * 2 + lax.axis_index("c")
        base = wid * per_w

        @pl.loop(0, iters)
        def _(i):
            b = base + i * chunk
            pltpu.sync_copy(idx_hbm.at[pl.ds(b, chunk)], idx_v)
            descs = [
                pltpu.async_copy(
                    table_hbm.at[idx_v.at[pl.ds(j * 128, 128)]],
                    rows_v.at[pl.ds(j * 128, 128)], sem)
                for j in range(nsub)
            ]
            for d in descs:
                d.wait()
            pltpu.sync_copy(rows_v, out_hbm.at[pl.ds(b, chunk)])

    return gk(table, idx)


def _sc_scatter_add(m, dst2d):
    rows_chunk = 1024
    groups = E_PAD // rows_chunk
    per_sub = groups // 16
    rows_per_sub = N_NODES // 16
    zrows = 125
    mesh = plsc.VectorSubcoreMesh(**_SC_MESH)

    @functools.partial(
        pl.kernel,
        out_type=jax.ShapeDtypeStruct((4, N_NODES, EMB), _f32),
        mesh=mesh,
        scratch_types=[
            pltpu.VMEM((8, 128), jnp.int32),
            pltpu.VMEM((rows_chunk, EMB), _f32),
            pltpu.VMEM((zrows, EMB), _f32),
            pltpu.VMEM_SHARED((N_NODES, EMB), _f32),
            pltpu.SemaphoreType.DMA,
        ],
        compiler_params=pltpu.CompilerParams(use_tc_tiling_on_sc=False),
    )
    def sk(m_hbm, d_hbm, out_hbm, idx_v, m_v, zbuf, acc, sem):
        c = lax.axis_index("c")
        sid = lax.axis_index("s")
        zero16 = jnp.zeros((16,), _f32)
        for r in range(zrows):
            zbuf[r, pl.ds(0, 16)] = zero16
        base_r = sid * rows_per_sub

        for sub in range(2):
            grp_c = c * 2 + sub

            @pl.loop(0, rows_per_sub // zrows)
            def _(r):
                pltpu.sync_copy(zbuf, acc.at[pl.ds(base_r + r * zrows, zrows)])

            plsc.subcore_barrier()

            @pl.loop(0, per_sub)
            def _(g):
                grp = sid * per_sub + g
                pltpu.sync_copy(
                    m_hbm.at[grp_c, pl.ds(grp * rows_chunk, rows_chunk)], m_v)
                pltpu.sync_copy(d_hbm.at[pl.ds(grp * 8, 8)], idx_v)
                for j in range(8):
                    pltpu.sync_copy(m_v.at[pl.ds(j * 128, 128)],
                                    acc.at[idx_v.at[j]], add=True)

            plsc.subcore_barrier()
            pltpu.sync_copy(acc.at[pl.ds(base_r, rows_per_sub)],
                            out_hbm.at[grp_c, pl.ds(base_r, rows_per_sub)])

    return sk(m, dst2d)


def _bdot(a, w, b):
    return lax.dot_general(a.astype(_bf16), w.astype(_bf16),
                           (((1,), (0,)), ((), ())),
                           preferred_element_type=_f32) + b[...]


def _embed_body(xf_ref, bb_ref, wemb_ref, bemb_ref, hs_ref):
    xf = xf_ref[...]
    aa = lax.broadcasted_iota(jnp.int32, (BN, 26), 1).astype(_f32)
    onehot = (aa == xf).astype(_f32)
    feat = jnp.concatenate([onehot, bb_ref[...]], axis=1)
    hs_ref[...] = _bdot(feat, wemb_ref[...], bemb_ref)


def _edge0_body(cs_ref, cd_ref, hs_ref, w1_ref, b1_ref, w2_ref, b2_ref,
                m_ref, ef_ref, y1_ref):
    i = pl.program_id(0)
    v = cs_ref[:, 0:4] - cd_ref[:, 0:4]
    len2 = jnp.sum(v * v, axis=1, keepdims=True)
    l = jnp.sqrt(len2)
    n = lax.broadcasted_iota(jnp.int32, (BE, 8), 1).astype(_f32) + 1.0
    arg = n * (jnp.pi / R_MAX) * l
    bes = jnp.sqrt(2.0 / R_MAX) * jnp.sin(arg) / l
    u = l * (1.0 / R_MAX)
    u2 = u * u
    u4 = u2 * u2
    u5 = u4 * u
    env = 1.0 - 21.0 * u5 + 35.0 * u5 * u - 15.0 * u5 * u2
    env = jnp.where(u < 1.0, env, 0.0)
    ef = bes * env
    ef_ref[...] = ef
    h1 = jnp.maximum(_bdot(ef, w1_ref[...], b1_ref), 0.0)
    w = _bdot(h1, w2_ref[...], b2_ref)
    linv = 1.0 / (l + 1e-12)
    y1 = jnp.sqrt(3.0) * v * linv
    y1_ref[...] = y1
    hs = hs_ref[...]
    eid = i * BE + lax.broadcasted_iota(jnp.int32, (BE, 1), 0)
    mask = eid < N_EDGES
    m_s = w[:, 0:16] * hs
    wh = w[:, 16:32] * hs
    zero = jnp.zeros_like(m_s)
    m_ref[0] = jnp.where(mask, m_s, zero)
    m_ref[1] = jnp.where(mask, wh * y1[:, 0:1], zero)
    m_ref[2] = jnp.where(mask, wh * y1[:, 1:2], zero)
    m_ref[3] = jnp.where(mask, wh * y1[:, 2:3], zero)


def _edge1_body(ef_ref, y1_ref, g_ref, w1_ref, b1_ref, w2_ref, b2_ref, m_ref):
    i = pl.program_id(0)
    ef = ef_ref[...]
    h1 = jnp.maximum(_bdot(ef, w1_ref[...], b1_ref), 0.0)
    w = _bdot(h1, w2_ref[...], b2_ref)
    g = g_ref[...]
    hs = g[:, 0:16]
    hvx = g[:, 16:32]
    hvy = g[:, 32:48]
    hvz = g[:, 48:64]
    y1 = y1_ref[...]
    y1x = y1[:, 0:1]
    y1y = y1[:, 1:2]
    y1z = y1[:, 2:3]
    dot = hvx * y1x + hvy * y1y + hvz * y1z
    m_s = w[:, 0:16] * hs + w[:, 48:64] * dot
    wh = w[:, 16:32] * hs
    m_vx = wh * y1x + w[:, 32:48] * hvx
    m_vy = wh * y1y + w[:, 32:48] * hvy
    m_vz = wh * y1z + w[:, 32:48] * hvz
    eid = i * BE + lax.broadcasted_iota(jnp.int32, (BE, 1), 0)
    mask = eid < N_EDGES
    zero = jnp.zeros_like(m_s)
    m_ref[0] = jnp.where(mask, m_s, zero)
    m_ref[1] = jnp.where(mask, m_vx, zero)
    m_ref[2] = jnp.where(mask, m_vy, zero)
    m_ref[3] = jnp.where(mask, m_vz, zero)


def _prod_body(with_v_res, agg_ref, res_ref, ws_ref, wv_ref, hall_ref):
    s = agg_ref[0]
    vx = agg_ref[1]
    vy = agg_ref[2]
    vz = agg_ref[3]
    vv = vx * vx + vy * vy + vz * vz
    s2 = s * s
    ws = ws_ref[...]
    wv = wv_ref[...]
    res = res_ref[...]
    out_s = (ws[0:1] * s + ws[1:2] * s2 + ws[2:3] * vv + ws[3:4] * (s2 * s)
             + ws[4:5] * (s * vv) + res[:, 0:16])
    coefv = wv[0:1] + wv[1:2] * s + wv[2:3] * vv + wv[3:4] * s2
    out_vx = coefv * vx
    out_vy = coefv * vy
    out_vz = coefv * vz
    if with_v_res:
        out_vx = out_vx + res[:, 16:32]
        out_vy = out_vy + res[:, 32:48]
        out_vz = out_vz + res[:, 48:64]
    hall_ref[...] = jnp.concatenate([out_s, out_vx, out_vy, out_vz], axis=1)


def _pool_body(hall_ref, bidx_ref, lw_ref, lb_ref, ow_ref, ob_ref,
               out_ref, acc_ref):
    i = pl.program_id(0)

    @pl.when(i == 0)
    def _():
        acc_ref[...] = jnp.zeros((N_GRAPHS, 4 * EMB), _f32)

    gids = lax.broadcasted_iota(jnp.int32, (BN, N_GRAPHS), 1).astype(_f32)
    onehot = (gids == bidx_ref[...]).astype(_f32)
    acc_ref[...] += lax.dot_general(onehot, hall_ref[...],
                                    (((0,), (0,)), ((), ())),
                                    preferred_element_type=_f32,
                                    precision=lax.Precision.HIGHEST)

    @pl.when(i == NGRID - 1)
    def _():
        pooled = acc_ref[...]
        h = jnp.maximum(_bdot(pooled, lw_ref[...], lb_ref), 0.0)
        out_ref[...] = _bdot(h, ow_ref[...], ob_ref)


def _full(shape):
    return pl.BlockSpec(shape, lambda i: tuple(0 for _ in shape))


def _tc_embed(xf, bb, wemb, bemb):
    return pl.pallas_call(
        _embed_body,
        grid=(NGRID,),
        in_specs=[
            pl.BlockSpec((BN, 1), lambda i: (i, 0)),
            pl.BlockSpec((BN, 6), lambda i: (i, 0)),
            _full((32, EMB)),
            _full((1, EMB)),
        ],
        out_specs=pl.BlockSpec((BN, EMB), lambda i: (i, 0)),
        out_shape=jax.ShapeDtypeStruct((N_NODES, EMB), _f32),
    )(xf, bb, wemb, bemb)


def _tc_edge0(cc, hs_src, w1, b1, w2, b2):
    return pl.pallas_call(
        _edge0_body,
        grid=(EGRID,),
        in_specs=[
            pl.BlockSpec((BE, 16), lambda i: (i, 0)),
            pl.BlockSpec((BE, 16), lambda i: (i + EGRID, 0)),
            pl.BlockSpec((BE, EMB), lambda i: (i, 0)),
            _full((8, 64)),
            _full((1, 64)),
            _full((64, 32)),
            _full((1, 32)),
        ],
        out_specs=[
            pl.BlockSpec((4, BE, EMB), lambda i: (0, i, 0)),
            pl.BlockSpec((BE, 8), lambda i: (i, 0)),
            pl.BlockSpec((BE, 4), lambda i: (i, 0)),
        ],
        out_shape=[
            jax.ShapeDtypeStruct((4, E_PAD, EMB), _f32),
            jax.ShapeDtypeStruct((E_PAD, 8), _f32),
            jax.ShapeDtypeStruct((E_PAD, 4), _f32),
        ],
    )(cc, cc, hs_src, w1, b1, w2, b2)


def _tc_edge1(ef, y1, g, w1, b1, w2, b2):
    return pl.pallas_call(
        _edge1_body,
        grid=(EGRID,),
        in_specs=[
            pl.BlockSpec((BE, 8), lambda i: (i, 0)),
            pl.BlockSpec((BE, 4), lambda i: (i, 0)),
            pl.BlockSpec((BE, 64), lambda i: (i, 0)),
            _full((8, 64)),
            _full((1, 64)),
            _full((64, 64)),
            _full((1, 64)),
        ],
        out_specs=pl.BlockSpec((4, BE, EMB), lambda i: (0, i, 0)),
        out_shape=jax.ShapeDtypeStruct((4, E_PAD, EMB), _f32),
    )(ef, y1, g, w1, b1, w2, b2)


def _tc_prod(agg, res, ws, wv, with_v_res):
    rdim = res.shape[1]
    return pl.pallas_call(
        functools.partial(_prod_body, with_v_res),
        grid=(NGRID,),
        in_specs=[
            pl.BlockSpec((4, BN, EMB), lambda i: (0, i, 0)),
            pl.BlockSpec((BN, rdim), lambda i: (i, 0)),
            _full((5, EMB)),
            _full((4, EMB)),
        ],
        out_specs=pl.BlockSpec((BN, 64), lambda i: (i, 0)),
        out_shape=jax.ShapeDtypeStruct((N_NODES, 64), _f32),
    )(agg, res, ws, wv)


def _tc_pool(hall, bidxf, lw, lb, ow, ob):
    nout = ow.shape[1]
    return pl.pallas_call(
        _pool_body,
        grid=(NGRID,),
        in_specs=[
            pl.BlockSpec((BN, 64), lambda i: (i, 0)),
            pl.BlockSpec((BN, 1), lambda i: (i, 0)),
            _full((64, 64)),
            _full((1, 64)),
            _full((64, nout)),
            _full((1, nout)),
        ],
        out_specs=pl.BlockSpec((N_GRAPHS, nout), lambda i: (0, 0)),
        out_shape=jax.ShapeDtypeStruct((N_GRAPHS, nout), _f32),
        scratch_shapes=[pltpu.VMEM((N_GRAPHS, 4 * EMB), _f32)],
    )(hall, bidxf, lw, lb, ow, ob)


def kernel(x, coords_ca, bb_embs, edge_index, batch_idx,
           W_emb, b_emb, mlp0_W1, mlp0_b1, mlp0_W2, mlp0_b2,
           mlp1_W1, mlp1_b1, mlp1_W2, mlp1_b2,
           ws0, wv0, ws1, wv1, lin1_W, lin1_b, out_W, out_b):
    pad = E_PAD - N_EDGES
    src = edge_index[0].astype(jnp.int32)
    dst = edge_index[1].astype(jnp.int32)
    zpad = jnp.zeros((pad,), jnp.int32)
    src_p = jnp.concatenate([src, zpad])
    dst_p = jnp.concatenate([dst, zpad])
    cidx = jnp.concatenate([src_p, dst_p])
    dst2d = dst_p.reshape(-1, 128)
    coords16 = jnp.pad(coords_ca, ((0, 0), (0, 13)))
    xf = x.astype(_f32).reshape(N_NODES, 1)
    bidxf = batch_idx.astype(_f32).reshape(N_NODES, 1)

    h_s = _tc_embed(xf, bb_embs, W_emb, b_emb.reshape(1, -1))
    cc = _sc_gather(coords16, cidx, 1024)
    hs_src = _sc_gather(h_s, src_p, 512)

    m0, ef, y1 = _tc_edge0(cc, hs_src,
                           mlp0_W1, mlp0_b1.reshape(1, -1),
                           mlp0_W2, mlp0_b2.reshape(1, -1))
    agg0 = _sc_scatter_add(m0, dst2d)
    hall1 = _tc_prod(agg0, h_s, ws0, wv0, with_v_res=False)

    g = _sc_gather(hall1, src_p, 512)
    m1 = _tc_edge1(ef, y1, g,
                   mlp1_W1, mlp1_b1.reshape(1, -1),
                   mlp1_W2, mlp1_b2.reshape(1, -1))
    agg1 = _sc_scatter_add(m1, dst2d)
    hall2 = _tc_prod(agg1, hall1, ws1, wv1, with_v_res=True)

    lin1_W = lin1_W[_LIN1_PERM]
    return _tc_pool(hall2, bidxf, lin1_W, lin1_b.reshape(1, -1),
                    out_W, out_b.reshape(1, -1))


# speedup vs baseline: 4.5701x; 1.2568x over previous
import functools

import numpy as np
import jax
import jax.numpy as jnp
from jax import lax
from jax.experimental import pallas as pl
from jax.experimental.pallas import tpu as pltpu
from jax.experimental.pallas import tpu_sc as plsc

N_NODES = 50000
N_PAD = 51200
N_EDGES = 800000
E_PAD = 802816
EMB = 16
R_MAX = 10.0
N_GRAPHS = 64

BE = 2048
EGRID = E_PAD // BE
BN = 2048
NGRID = N_PAD // BN

_f32 = jnp.float32
_bf16 = jnp.bfloat16

_SC_MESH = dict(core_axis_name="c", subcore_axis_name="s",
                num_cores=2, num_subcores=16)

_LIN1_PERM = np.concatenate([
    np.arange(16),
    np.array([16 + 3 * c + p for p in range(3) for c in range(16)]),
])


def _unpack(xp, n, w):
    return jnp.concatenate([xp[:, w * i:w * (i + 1)] for i in range(n)],
                           axis=0)


def _pack(xc, n):
    r = xc.shape[0] // n
    return jnp.concatenate([xc[r * i:r * (i + 1), :] for i in range(n)],
                           axis=1)


def _perm_edges(a):
    return a.reshape(-1, 8, BE // 8).transpose(0, 2, 1).reshape(-1)


def _sc_gather(table, idx, chunk):
    B = idx.shape[0]
    D = table.shape[1]
    per_w = B // 32
    iters = per_w // chunk
    nsub = chunk // 128
    mesh = plsc.VectorSubcoreMesh(**_SC_MESH)

    @functools.partial(
        pl.kernel,
        out_type=jax.ShapeDtypeStruct((B, D), _f32),
        mesh=mesh,
        scratch_types=[
            pltpu.VMEM((chunk,), jnp.int32),
            pltpu.VMEM((chunk, D), _f32),
            pltpu.SemaphoreType.DMA,
        ],
        compiler_params=pltpu.CompilerParams(use_tc_tiling_on_sc=False),
    )
    def gk(table_hbm, idx_hbm, out_hbm, idx_v, rows_v, sem):
        wid = lax.axis_index("s") * 2 + lax.axis_index("c")
        base = wid * per_w

        @pl.loop(0, iters)
        def _(i):
            b = base + i * chunk
            pltpu.sync_copy(idx_hbm.at[pl.ds(b, chunk)], idx_v)
            descs = [
                pltpu.async_copy(
                    table_hbm.at[idx_v.at[pl.ds(j * 128, 128)]],
                    rows_v.at[pl.ds(j * 128, 128)], sem)
                for j in range(nsub)
            ]
            for d in descs:
                d.wait()
            pltpu.sync_copy(rows_v, out_hbm.at[pl.ds(b, chunk)])

    return gk(table, idx)


def _sc_gather4(tables, idx, chunk):
    B = idx.shape[0]
    D = tables.shape[2]
    per_w = B // 32
    iters = per_w // chunk
    nsub = chunk // 128
    mesh = plsc.VectorSubcoreMesh(**_SC_MESH)

    @functools.partial(
        pl.kernel,
        out_type=jax.ShapeDtypeStruct((4, B, D), _f32),
        mesh=mesh,
        scratch_types=[
            pltpu.VMEM((chunk,), jnp.int32),
            pltpu.VMEM((4, chunk, D), _f32),
            pltpu.SemaphoreType.DMA,
        ],
        compiler_params=pltpu.CompilerParams(use_tc_tiling_on_sc=False),
    )
    def gk(tab_hbm, idx_hbm, out_hbm, idx_v, rows_v, sem):
        wid = lax.axis_index("s") * 2 + lax.axis_index("c")
        base = wid * per_w

        @pl.loop(0, iters)
        def _(i):
            b = base + i * chunk
            pltpu.sync_copy(idx_hbm.at[pl.ds(b, chunk)], idx_v)
            descs = [
                pltpu.async_copy(
                    tab_hbm.at[k].at[idx_v.at[pl.ds(j * 128, 128)]],
                    rows_v.at[k].at[pl.ds(j * 128, 128)], sem)
                for k in range(4)
                for j in range(nsub)
            ]
            for d in descs:
                d.wait()
            for k in range(4):
                pltpu.sync_copy(rows_v.at[k], out_hbm.at[k, pl.ds(b, chunk)])

    return gk(tables, idx)


def _sc_scatter_add(m, dst2d):
    rows_chunk = 1024
    groups = E_PAD // rows_chunk
    per_sub = groups // 16
    rows_per_sub = N_PAD // 16
    zrows = 128
    mesh = plsc.VectorSubcoreMesh(**_SC_MESH)

    @functools.partial(
        pl.kernel,
        out_type=jax.ShapeDtypeStruct((4, N_PAD, EMB), _f32),
        mesh=mesh,
        scratch_types=[
            pltpu.VMEM((8, 128), jnp.int32),
            pltpu.VMEM((rows_chunk, EMB), _f32),
            pltpu.VMEM((zrows, EMB), _f32),
            pltpu.VMEM_SHARED((N_PAD, EMB), _f32),
            pltpu.SemaphoreType.DMA,
        ],
        compiler_params=pltpu.CompilerParams(use_tc_tiling_on_sc=False),
    )
    def sk(m_hbm, d_hbm, out_hbm, idx_v, m_v, zbuf, acc, sem):
        c = lax.axis_index("c")
        sid = lax.axis_index("s")
        zero16 = jnp.zeros((16,), _f32)
        for r in range(zrows):
            zbuf[r, pl.ds(0, 16)] = zero16
        base_r = sid * rows_per_sub

        for sub in range(2):
            grp_c = c * 2 + sub

            @pl.loop(0, rows_per_sub // zrows)
            def _(r):
                pltpu.sync_copy(zbuf, acc.at[pl.ds(base_r + r * zrows, zrows)])

            plsc.subcore_barrier()

            @pl.loop(0, per_sub)
            def _(g):
                grp = sid * per_sub + g
                pltpu.sync_copy(
                    m_hbm.at[grp_c, pl.ds(grp * rows_chunk, rows_chunk)], m_v)
                pltpu.sync_copy(d_hbm.at[pl.ds(grp * 8, 8)], idx_v)
                for j in range(8):
                    pltpu.sync_copy(m_v.at[pl.ds(j * 128, 128)],
                                    acc.at[idx_v.at[j]], add=True)

            plsc.subcore_barrier()
            pltpu.sync_copy(acc.at[pl.ds(base_r, rows_per_sub)],
                            out_hbm.at[grp_c, pl.ds(base_r, rows_per_sub)])

    return sk(m, dst2d)


def _bdot(a, w, b):
    return lax.dot_general(a.astype(_bf16), w.astype(_bf16),
                           (((1,), (0,)), ((), ())),
                           preferred_element_type=_f32) + b[...]


def _embed_body(xfp_ref, bbp_ref, wemb_ref, bemb_ref, hp_ref):
    xf = _unpack(xfp_ref[...], 8, 1)
    bb = _unpack(bbp_ref[...], 8, 6)
    aa = lax.broadcasted_iota(jnp.int32, (BN, 26), 1).astype(_f32)
    onehot = (aa == xf).astype(_f32)
    feat = jnp.concatenate([onehot, bb], axis=1)
    hp_ref[...] = _pack(_bdot(feat, wemb_ref[...], bemb_ref), 8)


def _edge0_body(csp_ref, cdp_ref, hsp_ref, w1_ref, b1_ref, w2_ref, b2_ref,
                m_ref, efyp_ref):
    i = pl.program_id(0)
    cs = _unpack(csp_ref[...], 8, 16)
    cd = _unpack(cdp_ref[...], 8, 16)
    v = cs[:, 0:4] - cd[:, 0:4]
    len2 = jnp.sum(v * v, axis=1, keepdims=True)
    l = jnp.sqrt(len2)
    n = lax.broadcasted_iota(jnp.int32, (BE, 8), 1).astype(_f32) + 1.0
    arg = n * (jnp.pi / R_MAX) * l
    bes = jnp.sqrt(2.0 / R_MAX) * jnp.sin(arg) / l
    u = l * (1.0 / R_MAX)
    u2 = u * u
    u4 = u2 * u2
    u5 = u4 * u
    env = 1.0 - 21.0 * u5 + 35.0 * u5 * u - 15.0 * u5 * u2
    env = jnp.where(u < 1.0, env, 0.0)
    ef = bes * env
    h1 = jnp.maximum(_bdot(ef, w1_ref[...], b1_ref), 0.0)
    w = _bdot(h1, w2_ref[...], b2_ref)
    linv = 1.0 / (l + 1e-12)
    y1 = jnp.sqrt(3.0) * v * linv
    efyp_ref[...] = _pack(
        jnp.concatenate([ef, y1, jnp.zeros((BE, 4), _f32)], axis=1), 8)
    hs = _unpack(hsp_ref[...], 8, 16)
    eid = i * BE + lax.broadcasted_iota(jnp.int32, (BE, 1), 0)
    mask = eid < N_EDGES
    m_s = w[:, 0:16] * hs
    wh = w[:, 16:32] * hs
    zero = jnp.zeros_like(m_s)
    m_ref[0] = _pack(jnp.where(mask, m_s, zero), 8)
    m_ref[1] = _pack(jnp.where(mask, wh * y1[:, 0:1], zero), 8)
    m_ref[2] = _pack(jnp.where(mask, wh * y1[:, 1:2], zero), 8)
    m_ref[3] = _pack(jnp.where(mask, wh * y1[:, 2:3], zero), 8)


def _edge1_body(efyp_ref, gp_ref, w1_ref, b1_ref, w2_ref, b2_ref, m_ref):
    i = pl.program_id(0)
    efy = _unpack(efyp_ref[...], 8, 16)
    ef = efy[:, 0:8]
    h1 = jnp.maximum(_bdot(ef, w1_ref[...], b1_ref), 0.0)
    w = _bdot(h1, w2_ref[...], b2_ref)
    hs = _unpack(gp_ref[0], 8, 16)
    hvx = _unpack(gp_ref[1], 8, 16)
    hvy = _unpack(gp_ref[2], 8, 16)
    hvz = _unpack(gp_ref[3], 8, 16)
    y1x = efy[:, 8:9]
    y1y = efy[:, 9:10]
    y1z = efy[:, 10:11]
    dot = hvx * y1x + hvy * y1y + hvz * y1z
    m_s = w[:, 0:16] * hs + w[:, 48:64] * dot
    wh = w[:, 16:32] * hs
    m_vx = wh * y1x + w[:, 32:48] * hvx
    m_vy = wh * y1y + w[:, 32:48] * hvy
    m_vz = wh * y1z + w[:, 32:48] * hvz
    eid = i * BE + lax.broadcasted_iota(jnp.int32, (BE, 1), 0)
    mask = eid < N_EDGES
    zero = jnp.zeros_like(m_s)
    m_ref[0] = _pack(jnp.where(mask, m_s, zero), 8)
    m_ref[1] = _pack(jnp.where(mask, m_vx, zero), 8)
    m_ref[2] = _pack(jnp.where(mask, m_vy, zero), 8)
    m_ref[3] = _pack(jnp.where(mask, m_vz, zero), 8)


def _prod_body(with_v_res, aggp_ref, resp_ref, ws_ref, wv_ref, hallp_ref):
    s = _unpack(aggp_ref[0], 8, 16)
    vx = _unpack(aggp_ref[1], 8, 16)
    vy = _unpack(aggp_ref[2], 8, 16)
    vz = _unpack(aggp_ref[3], 8, 16)
    vv = vx * vx + vy * vy + vz * vz
    s2 = s * s
    ws = ws_ref[...]
    wv = wv_ref[...]
    out_s = (ws[0:1] * s + ws[1:2] * s2 + ws[2:3] * vv + ws[3:4] * (s2 * s)
             + ws[4:5] * (s * vv))
    coefv = wv[0:1] + wv[1:2] * s + wv[2:3] * vv + wv[3:4] * s2
    out_vx = coefv * vx
    out_vy = coefv * vy
    out_vz = coefv * vz
    if with_v_res:
        out_s = out_s + _unpack(resp_ref[0], 8, 16)
        out_vx = out_vx + _unpack(resp_ref[1], 8, 16)
        out_vy = out_vy + _unpack(resp_ref[2], 8, 16)
        out_vz = out_vz + _unpack(resp_ref[3], 8, 16)
    else:
        out_s = out_s + _unpack(resp_ref[...], 8, 16)
    hallp_ref[0] = _pack(out_s, 8)
    hallp_ref[1] = _pack(out_vx, 8)
    hallp_ref[2] = _pack(out_vy, 8)
    hallp_ref[3] = _pack(out_vz, 8)


def _pool_body(hallp_ref, bidxp_ref, lw_ref, lb_ref, ow_ref, ob_ref,
               out_ref, acc_ref):
    i = pl.program_id(0)

    @pl.when(i == 0)
    def _():
        acc_ref[...] = jnp.zeros((N_GRAPHS, 4 * EMB), _f32)

    h = jnp.concatenate([_unpack(hallp_ref[k], 8, 16) for k in range(4)],
                        axis=1)
    bidx = _unpack(bidxp_ref[...], 8, 1)
    gids = lax.broadcasted_iota(jnp.int32, (BN, N_GRAPHS), 1).astype(_f32)
    onehot = (gids == bidx).astype(_f32)
    acc_ref[...] += lax.dot_general(onehot, h, (((0,), (0,)), ((), ())),
                                    preferred_element_type=_f32,
                                    precision=lax.Precision.HIGHEST)

    @pl.when(i == NGRID - 1)
    def _():
        pooled = acc_ref[...]
        hmid = jnp.maximum(_bdot(pooled, lw_ref[...], lb_ref), 0.0)
        out_ref[...] = _bdot(hmid, ow_ref[...], ob_ref)


def _full(shape):
    return pl.BlockSpec(shape, lambda i: tuple(0 for _ in shape))


def _tc_embed(xfp, bbp, wemb, bemb):
    return pl.pallas_call(
        _embed_body,
        grid=(NGRID,),
        in_specs=[
            pl.BlockSpec((BN // 8, 8), lambda i: (i, 0)),
            pl.BlockSpec((BN // 8, 48), lambda i: (i, 0)),
            _full((32, EMB)),
            _full((1, EMB)),
        ],
        out_specs=pl.BlockSpec((BN // 8, 128), lambda i: (i, 0)),
        out_shape=jax.ShapeDtypeStruct((N_PAD // 8, 128), _f32),
    )(xfp, bbp, wemb, bemb)


def _tc_edge0(ccp, hsp, w1, b1, w2, b2):
    return pl.pallas_call(
        _edge0_body,
        grid=(EGRID,),
        in_specs=[
            pl.BlockSpec((BE // 8, 128), lambda i: (i, 0)),
            pl.BlockSpec((BE // 8, 128), lambda i: (i + EGRID, 0)),
            pl.BlockSpec((BE // 8, 128), lambda i: (i, 0)),
            _full((8, 64)),
            _full((1, 64)),
            _full((64, 32)),
            _full((1, 32)),
        ],
        out_specs=[
            pl.BlockSpec((4, BE // 8, 128), lambda i: (0, i, 0)),
            pl.BlockSpec((BE // 8, 128), lambda i: (i, 0)),
        ],
        out_shape=[
            jax.ShapeDtypeStruct((4, E_PAD // 8, 128), _f32),
            jax.ShapeDtypeStruct((E_PAD // 8, 128), _f32),
        ],
    )(ccp, ccp, hsp, w1, b1, w2, b2)


def _tc_edge1(efyp, gp, w1, b1, w2, b2):
    return pl.pallas_call(
        _edge1_body,
        grid=(EGRID,),
        in_specs=[
            pl.BlockSpec((BE // 8, 128), lambda i: (i, 0)),
            pl.BlockSpec((4, BE // 8, 128), lambda i: (0, i, 0)),
            _full((8, 64)),
            _full((1, 64)),
            _full((64, 64)),
            _full((1, 64)),
        ],
        out_specs=pl.BlockSpec((4, BE // 8, 128), lambda i: (0, i, 0)),
        out_shape=jax.ShapeDtypeStruct((4, E_PAD // 8, 128), _f32),
    )(efyp, gp, w1, b1, w2, b2)


def _tc_prod(aggp, resp, ws, wv, with_v_res):
    if with_v_res:
        res_spec = pl.BlockSpec((4, BN // 8, 128), lambda i: (0, i, 0))
    else:
        res_spec = pl.BlockSpec((BN // 8, 128), lambda i: (i, 0))
    return pl.pallas_call(
        functools.partial(_prod_body, with_v_res),
        grid=(NGRID,),
        in_specs=[
            pl.BlockSpec((4, BN // 8, 128), lambda i: (0, i, 0)),
            res_spec,
            _full((5, EMB)),
            _full((4, EMB)),
        ],
        out_specs=pl.BlockSpec((4, BN // 8, 128), lambda i: (0, i, 0)),
        out_shape=jax.ShapeDtypeStruct((4, N_PAD // 8, 128), _f32),
    )(aggp, resp, ws, wv)


def _tc_pool(hallp, bidxp, lw, lb, ow, ob):
    nout = ow.shape[1]
    return pl.pallas_call(
        _pool_body,
        grid=(NGRID,),
        in_specs=[
            pl.BlockSpec((4, BN // 8, 128), lambda i: (0, i, 0)),
            pl.BlockSpec((BN // 8, 8), lambda i: (i, 0)),
            _full((64, 64)),
            _full((1, 64)),
            _full((64, nout)),
            _full((1, nout)),
        ],
        out_specs=pl.BlockSpec((N_GRAPHS, nout), lambda i: (0, 0)),
        out_shape=jax.ShapeDtypeStruct((N_GRAPHS, nout), _f32),
        scratch_shapes=[pltpu.VMEM((N_GRAPHS, 4 * EMB), _f32)],
    )(hallp, bidxp, lw, lb, ow, ob)


def kernel(x, coords_ca, bb_embs, edge_index, batch_idx,
           W_emb, b_emb, mlp0_W1, mlp0_b1, mlp0_W2, mlp0_b2,
           mlp1_W1, mlp1_b1, mlp1_W2, mlp1_b2,
           ws0, wv0, ws1, wv1, lin1_W, lin1_b, out_W, out_b):
    pad = E_PAD - N_EDGES
    zpad = jnp.zeros((pad,), jnp.int32)
    src_p = _perm_edges(jnp.concatenate([edge_index[0].astype(jnp.int32),
                                         zpad]))
    dst_p = _perm_edges(jnp.concatenate([edge_index[1].astype(jnp.int32),
                                         zpad]))
    cidx = jnp.concatenate([src_p, dst_p])
    dst2d = dst_p.reshape(-1, 128)
    coords16 = jnp.pad(coords_ca, ((0, 0), (0, 13)))
    npad = N_PAD - N_NODES
    xfp = jnp.concatenate(
        [x.astype(_f32), jnp.full((npad,), -1.0, _f32)]).reshape(N_PAD // 8, 8)
    bbp = jnp.pad(bb_embs, ((0, npad), (0, 0))).reshape(N_PAD // 8, 48)
    bidxp = jnp.concatenate(
        [batch_idx.astype(_f32),
         jnp.full((npad,), -1.0, _f32)]).reshape(N_PAD // 8, 8)

    hp = _tc_embed(xfp, bbp, W_emb, b_emb.reshape(1, -1))
    ccp = _sc_gather(coords16, cidx, 1024).reshape(-1, 128)
    hsp = _sc_gather(hp.reshape(N_PAD, EMB), src_p, 512).reshape(-1, 128)

    m0, efyp = _tc_edge0(ccp, hsp,
                         mlp0_W1, mlp0_b1.reshape(1, -1),
                         mlp0_W2, mlp0_b2.reshape(1, -1))
    agg0 = _sc_scatter_add(m0.reshape(4, E_PAD, EMB),
                           dst2d).reshape(4, N_PAD // 8, 128)
    hall1 = _tc_prod(agg0, hp, ws0, wv0, with_v_res=False)

    gp = _sc_gather4(hall1.reshape(4, N_PAD, EMB),
                     src_p, 512).reshape(4, E_PAD // 8, 128)
    m1 = _tc_edge1(efyp, gp,
                   mlp1_W1, mlp1_b1.reshape(1, -1),
                   mlp1_W2, mlp1_b2.reshape(1, -1))
    agg1 = _sc_scatter_add(m1.reshape(4, E_PAD, EMB),
                           dst2d).reshape(4, N_PAD // 8, 128)
    hall2 = _tc_prod(agg1, hall1, ws1, wv1, with_v_res=True)

    return _tc_pool(hall2, bidxp, lin1_W[_LIN1_PERM], lin1_b.reshape(1, -1),
                    out_W, out_b.reshape(1, -1))


# speedup vs baseline: 10.5797x; 2.3150x over previous
import functools

import numpy as np
import jax
import jax.numpy as jnp
from jax import lax
from jax.experimental import pallas as pl
from jax.experimental.pallas import tpu as pltpu
from jax.experimental.pallas import tpu_sc as plsc

N_NODES = 50000
N_PAD = 51200
N_EDGES = 800000
E_PAD = 802816
EMB = 16
R_MAX = 10.0
N_GRAPHS = 64

BE = 2048
EGRID = E_PAD // BE
BN = 2048
NGRID = N_PAD // BN

_f32 = jnp.float32
_bf16 = jnp.bfloat16

_SC_MESH = dict(core_axis_name="c", subcore_axis_name="s",
                num_cores=2, num_subcores=16)

_LIN1_PERM = np.concatenate([
    np.arange(16),
    np.array([16 + 3 * c + p for p in range(3) for c in range(16)]),
])


def _unpack(xp, n, w):
    return jnp.concatenate([xp[:, w * i:w * (i + 1)] for i in range(n)],
                           axis=0)


def _pack(xc, n):
    r = xc.shape[0] // n
    return jnp.concatenate([xc[r * i:r * (i + 1), :] for i in range(n)],
                           axis=1)


def _perm_edges(a):
    return a.reshape(-1, 8, BE // 8).transpose(0, 2, 1).reshape(-1)


def _np_sel(shape, entries):
    m = np.zeros(shape, np.float32)
    for r, c, v in entries:
        m[r, c] = v
    return m


_P3 = _np_sel((128, 8), [(16 * i + j, i, 1.0) for i in range(8) for j in range(3)])
_PX = _np_sel((128, 8), [(16 * i, i, 1.0) for i in range(8)])
_PY = _np_sel((128, 8), [(16 * i + 1, i, 1.0) for i in range(8)])
_PZ = _np_sel((128, 8), [(16 * i + 2, i, 1.0) for i in range(8)])
_Q8 = _np_sel((8, 64), [(i, 8 * i + f, 1.0) for i in range(8) for f in range(8)])
_Q8N = _np_sel((8, 64), [(i, 8 * i + f, (f + 1) * np.pi / R_MAX)
                         for i in range(8) for f in range(8)])
_Q16 = _np_sel((8, 128), [(i, 16 * i + j, 1.0)
                          for i in range(8) for j in range(16)])
_QYX = _np_sel((32, 128), [(i, 16 * i + j, 1.0)
                           for i in range(8) for j in range(16)])
_QYY = _np_sel((32, 128), [(8 + i, 16 * i + j, 1.0)
                           for i in range(8) for j in range(16)])
_QYZ = _np_sel((32, 128), [(16 + i, 16 * i + j, 1.0)
                           for i in range(8) for j in range(16)])
_QZ = _np_sel((32, 128), [(24 + i, 16 * i + j, 1.0)
                          for i in range(8) for j in range(16)])


def _sc_gather(table, idx, chunk):
    B = idx.shape[0]
    D = table.shape[1]
    per_w = B // 32
    iters = per_w // chunk
    nsub = chunk // 128
    mesh = plsc.VectorSubcoreMesh(**_SC_MESH)

    @functools.partial(
        pl.kernel,
        out_type=jax.ShapeDtypeStruct((B, D), _f32),
        mesh=mesh,
        scratch_types=[
            pltpu.VMEM((chunk,), jnp.int32),
            pltpu.VMEM((chunk, D), _f32),
            pltpu.SemaphoreType.DMA,
        ],
        compiler_params=pltpu.CompilerParams(use_tc_tiling_on_sc=False),
    )
    def gk(table_hbm, idx_hbm, out_hbm, idx_v, rows_v, sem):
        wid = lax.axis_index("s") * 2 + lax.axis_index("c")
        base = wid * per_w

        @pl.loop(0, iters)
        def _(i):
            b = base + i * chunk
            pltpu.sync_copy(idx_hbm.at[pl.ds(b, chunk)], idx_v)
            descs = [
                pltpu.async_copy(
                    table_hbm.at[idx_v.at[pl.ds(j * 128, 128)]],
                    rows_v.at[pl.ds(j * 128, 128)], sem)
                for j in range(nsub)
            ]
            for d in descs:
                d.wait()
            pltpu.sync_copy(rows_v, out_hbm.at[pl.ds(b, chunk)])

    return gk(table, idx)


def _sc_gather4(tables, idx, chunk):
    B = idx.shape[0]
    D = tables.shape[2]
    per_w = B // 32
    iters = per_w // chunk
    nsub = chunk // 128
    mesh = plsc.VectorSubcoreMesh(**_SC_MESH)

    @functools.partial(
        pl.kernel,
        out_type=jax.ShapeDtypeStruct((4, B, D), _f32),
        mesh=mesh,
        scratch_types=[
            pltpu.VMEM((chunk,), jnp.int32),
            pltpu.VMEM((4, chunk, D), _f32),
            pltpu.SemaphoreType.DMA,
        ],
        compiler_params=pltpu.CompilerParams(use_tc_tiling_on_sc=False),
    )
    def gk(tab_hbm, idx_hbm, out_hbm, idx_v, rows_v, sem):
        wid = lax.axis_index("s") * 2 + lax.axis_index("c")
        base = wid * per_w

        @pl.loop(0, iters)
        def _(i):
            b = base + i * chunk
            pltpu.sync_copy(idx_hbm.at[pl.ds(b, chunk)], idx_v)
            descs = [
                pltpu.async_copy(
                    tab_hbm.at[k].at[idx_v.at[pl.ds(j * 128, 128)]],
                    rows_v.at[k].at[pl.ds(j * 128, 128)], sem)
                for k in range(4)
                for j in range(nsub)
            ]
            for d in descs:
                d.wait()
            for k in range(4):
                pltpu.sync_copy(rows_v.at[k], out_hbm.at[k, pl.ds(b, chunk)])

    return gk(tables, idx)


def _sc_scatter_add(m, dst2d):
    rows_chunk = 1024
    groups = E_PAD // rows_chunk
    per_sub = groups // 16
    rows_per_sub = N_PAD // 16
    zrows = 128
    mesh = plsc.VectorSubcoreMesh(**_SC_MESH)

    @functools.partial(
        pl.kernel,
        out_type=jax.ShapeDtypeStruct((4, N_PAD, EMB), _f32),
        mesh=mesh,
        scratch_types=[
            pltpu.VMEM((8, 128), jnp.int32),
            pltpu.VMEM((rows_chunk, EMB), _f32),
            pltpu.VMEM((zrows, EMB), _f32),
            pltpu.VMEM_SHARED((N_PAD, EMB), _f32),
            pltpu.SemaphoreType.DMA,
        ],
        compiler_params=pltpu.CompilerParams(use_tc_tiling_on_sc=False),
    )
    def sk(m_hbm, d_hbm, out_hbm, idx_v, m_v, zbuf, acc, sem):
        c = lax.axis_index("c")
        sid = lax.axis_index("s")
        zero16 = jnp.zeros((16,), _f32)
        for r in range(zrows):
            zbuf[r, pl.ds(0, 16)] = zero16
        base_r = sid * rows_per_sub

        for sub in range(2):
            grp_c = c * 2 + sub

            @pl.loop(0, rows_per_sub // zrows)
            def _(r):
                pltpu.sync_copy(zbuf, acc.at[pl.ds(base_r + r * zrows, zrows)])

            plsc.subcore_barrier()

            @pl.loop(0, per_sub)
            def _(g):
                grp = sid * per_sub + g
                pltpu.sync_copy(
                    m_hbm.at[grp_c, pl.ds(grp * rows_chunk, rows_chunk)], m_v)
                pltpu.sync_copy(d_hbm.at[pl.ds(grp * 8, 8)], idx_v)
                for j in range(8):
                    pltpu.sync_copy(m_v.at[pl.ds(j * 128, 128)],
                                    acc.at[idx_v.at[j]], add=True)

            plsc.subcore_barrier()
            pltpu.sync_copy(acc.at[pl.ds(base_r, rows_per_sub)],
                            out_hbm.at[grp_c, pl.ds(base_r, rows_per_sub)])

    return sk(m, dst2d)


def _bdot(a, w, b):
    return lax.dot_general(a.astype(_bf16), w.astype(_bf16),
                           (((1,), (0,)), ((), ())),
                           preferred_element_type=_f32) + b[...]


def _hd(a, m):
    return lax.dot_general(a, m, (((1,), (0,)), ((), ())),
                           preferred_element_type=_f32,
                           precision=lax.Precision.HIGHEST)


def _embed_body(xfp_ref, bbp_ref, wemb_ref, bemb_ref, hp_ref):
    xf = _unpack(xfp_ref[...], 8, 1)
    bb = _unpack(bbp_ref[...], 8, 6)
    aa = lax.broadcasted_iota(jnp.int32, (BN, 26), 1).astype(_f32)
    onehot = (aa == xf).astype(_f32)
    feat = jnp.concatenate([onehot, bb], axis=1)
    hp_ref[...] = _pack(_bdot(feat, wemb_ref[...], bemb_ref), 8)


def _edge_mask(i):
    row = lax.broadcasted_iota(jnp.int32, (BE // 8, 128), 0)
    lane = lax.broadcasted_iota(jnp.int32, (BE // 8, 128), 1)
    eid = i * BE + (BE // 8) * lax.shift_right_logical(lane, 4) + row
    return eid < N_EDGES


def _emit_m(m_ref, k, val, valid, z16):
    nanv = jnp.full_like(val, jnp.nan)
    poisoned = jnp.where(z16 > 0.0, nanv, val)
    m_ref[k] = jnp.where(valid, poisoned, jnp.zeros_like(val))


def _edge0_body(csp_ref, cdp_ref, hsp_ref, p3_ref, px_ref, py_ref, pz_ref,
                q8_ref, q8n_ref, q16_ref, w1_ref, b1_ref, w2_ref, b2_ref,
                m_ref, ef_ref, y_ref):
    i = pl.program_id(0)
    d = csp_ref[...] - cdp_ref[...]
    len2 = _hd(d * d, p3_ref[...])
    l8 = jnp.sqrt(len2)
    u = l8 * (1.0 / R_MAX)
    u2 = u * u
    u5 = u2 * u2 * u
    env8 = jnp.where(u < 1.0,
                     1.0 - 21.0 * u5 + 35.0 * u5 * u - 15.0 * u5 * u2, 0.0)
    linv8 = 1.0 / (l8 + 1e-12)
    q8 = q8_ref[...]
    arg = _hd(l8, q8n_ref[...])
    l64 = _hd(l8, q8)
    bes = jnp.sqrt(2.0 / R_MAX) * jnp.sin(arg) / l64
    ef = bes * _hd(env8, q8)
    z8 = (l8 == 0.0).astype(_f32)
    z64 = _hd(z8, q8)
    ef_clean = jnp.where(z64 > 0.0, 0.0, ef)
    ef_ref[...] = ef_clean
    sq3 = jnp.sqrt(3.0)
    y1x8 = sq3 * _hd(d, px_ref[...]) * linv8
    y1y8 = sq3 * _hd(d, py_ref[...]) * linv8
    y1z8 = sq3 * _hd(d, pz_ref[...]) * linv8
    y_ref[...] = jnp.concatenate([y1x8, y1y8, y1z8, z8], axis=1)
    h1 = jnp.maximum(_bdot(ef_clean, w1_ref[...], b1_ref), 0.0)
    w_all = _bdot(h1, w2_ref[...], b2_ref)
    wp0 = w_all[:, 0:128]
    wp1 = w_all[:, 128:256]
    hs = hsp_ref[...]
    q16 = q16_ref[...]
    z16 = _hd(z8, q16)
    valid = _edge_mask(i)
    wh = wp1 * hs
    _emit_m(m_ref, 0, wp0 * hs, valid, z16)
    _emit_m(m_ref, 1, wh * _hd(y1x8, q16), valid, z16)
    _emit_m(m_ref, 2, wh * _hd(y1y8, q16), valid, z16)
    _emit_m(m_ref, 3, wh * _hd(y1z8, q16), valid, z16)


def _edge1_body(ef_ref, y_ref, gp_ref, qyx_ref, qyy_ref, qyz_ref, qz_ref,
                w1_ref, b1_ref, w2_ref, b2_ref, m_ref):
    i = pl.program_id(0)
    ef = ef_ref[...]
    y = y_ref[...]
    h1 = jnp.maximum(_bdot(ef, w1_ref[...], b1_ref), 0.0)
    w_all = _bdot(h1, w2_ref[...], b2_ref)
    wp0 = w_all[:, 0:128]
    wp1 = w_all[:, 128:256]
    wp2 = w_all[:, 256:384]
    wp3 = w_all[:, 384:512]
    y1x = _hd(y, qyx_ref[...])
    y1y = _hd(y, qyy_ref[...])
    y1z = _hd(y, qyz_ref[...])
    z16 = _hd(y, qz_ref[...])
    hs = gp_ref[0]
    hvx = gp_ref[1]
    hvy = gp_ref[2]
    hvz = gp_ref[3]
    dot = hvx * y1x + hvy * y1y + hvz * y1z
    valid = _edge_mask(i)
    wh = wp1 * hs
    _emit_m(m_ref, 0, wp0 * hs + wp3 * dot, valid, z16)
    _emit_m(m_ref, 1, wh * y1x + wp2 * hvx, valid, z16)
    _emit_m(m_ref, 2, wh * y1y + wp2 * hvy, valid, z16)
    _emit_m(m_ref, 3, wh * y1z + wp2 * hvz, valid, z16)


def _prod_body(with_v_res, aggp_ref, resp_ref, ws_ref, wv_ref, hallp_ref):
    s = _unpack(aggp_ref[0], 8, 16)
    vx = _unpack(aggp_ref[1], 8, 16)
    vy = _unpack(aggp_ref[2], 8, 16)
    vz = _unpack(aggp_ref[3], 8, 16)
    vv = vx * vx + vy * vy + vz * vz
    s2 = s * s
    ws = ws_ref[...]
    wv = wv_ref[...]
    out_s = (ws[0:1] * s + ws[1:2] * s2 + ws[2:3] * vv + ws[3:4] * (s2 * s)
             + ws[4:5] * (s * vv))
    coefv = wv[0:1] + wv[1:2] * s + wv[2:3] * vv + wv[3:4] * s2
    out_vx = coefv * vx
    out_vy = coefv * vy
    out_vz = coefv * vz
    if with_v_res:
        out_s = out_s + _unpack(resp_ref[0], 8, 16)
        out_vx = out_vx + _unpack(resp_ref[1], 8, 16)
        out_vy = out_vy + _unpack(resp_ref[2], 8, 16)
        out_vz = out_vz + _unpack(resp_ref[3], 8, 16)
    else:
        out_s = out_s + _unpack(resp_ref[...], 8, 16)
    hallp_ref[0] = _pack(out_s, 8)
    hallp_ref[1] = _pack(out_vx, 8)
    hallp_ref[2] = _pack(out_vy, 8)
    hallp_ref[3] = _pack(out_vz, 8)


def _pool_body(hallp_ref, bidxp_ref, lw_ref, lb_ref, ow_ref, ob_ref,
               out_ref, acc_ref):
    i = pl.program_id(0)

    @pl.when(i == 0)
    def _():
        acc_ref[...] = jnp.zeros((N_GRAPHS, 8 * EMB), _f32)

    h = jnp.concatenate([_unpack(hallp_ref[k], 8, 16) for k in range(4)],
                        axis=1)
    isn = jnp.sum(jnp.where(jnp.isnan(h), 1.0, 0.0), axis=1, keepdims=True)
    flag = jnp.where(isn > 0.0, 1.0, 0.0)
    h_clean = jnp.where(isn > 0.0, 0.0, h)
    h_ext = jnp.concatenate(
        [h_clean, jnp.broadcast_to(flag, (BN, 4 * EMB))], axis=1)
    bidx = _unpack(bidxp_ref[...], 8, 1)
    gids = lax.broadcasted_iota(jnp.int32, (BN, N_GRAPHS), 1).astype(_f32)
    onehot = (gids == bidx).astype(_f32)
    acc_ref[...] += lax.dot_general(onehot, h_ext, (((0,), (0,)), ((), ())),
                                    preferred_element_type=_f32,
                                    precision=lax.Precision.HIGHEST)

    @pl.when(i == NGRID - 1)
    def _():
        pooled = acc_ref[:, 0:4 * EMB]
        cnt = acc_ref[:, 4 * EMB:4 * EMB + 1]
        hmid = jnp.maximum(_bdot(pooled, lw_ref[...], lb_ref), 0.0)
        out = _bdot(hmid, ow_ref[...], ob_ref)
        out_ref[...] = jnp.where(cnt > 0.0, jnp.full_like(out, jnp.nan), out)


def _full(shape):
    return pl.BlockSpec(shape, lambda i: tuple(0 for _ in shape))


def _tc_embed(xfp, bbp, wemb, bemb):
    return pl.pallas_call(
        _embed_body,
        grid=(NGRID,),
        in_specs=[
            pl.BlockSpec((BN // 8, 8), lambda i: (i, 0)),
            pl.BlockSpec((BN // 8, 48), lambda i: (i, 0)),
            _full((32, EMB)),
            _full((1, EMB)),
        ],
        out_specs=pl.BlockSpec((BN // 8, 128), lambda i: (i, 0)),
        out_shape=jax.ShapeDtypeStruct((N_PAD // 8, 128), _f32),
    )(xfp, bbp, wemb, bemb)


def _tc_edge0(ccp, hsp, bd1, b1t, bd2, b2t):
    return pl.pallas_call(
        _edge0_body,
        grid=(EGRID,),
        in_specs=[
            pl.BlockSpec((BE // 8, 128), lambda i: (i, 0)),
            pl.BlockSpec((BE // 8, 128), lambda i: (i + EGRID, 0)),
            pl.BlockSpec((BE // 8, 128), lambda i: (i, 0)),
            _full((128, 8)),
            _full((128, 8)),
            _full((128, 8)),
            _full((128, 8)),
            _full((8, 64)),
            _full((8, 64)),
            _full((8, 128)),
            _full((64, 512)),
            _full((1, 512)),
            _full((512, 256)),
            _full((1, 256)),
        ],
        out_specs=[
            pl.BlockSpec((4, BE // 8, 128), lambda i: (0, i, 0)),
            pl.BlockSpec((BE // 8, 64), lambda i: (i, 0)),
            pl.BlockSpec((BE // 8, 32), lambda i: (i, 0)),
        ],
        out_shape=[
            jax.ShapeDtypeStruct((4, E_PAD // 8, 128), _f32),
            jax.ShapeDtypeStruct((E_PAD // 8, 64), _f32),
            jax.ShapeDtypeStruct((E_PAD // 8, 32), _f32),
        ],
    )(ccp, ccp, hsp, _P3, _PX, _PY, _PZ, _Q8, _Q8N, _Q16,
      bd1, b1t, bd2, b2t)


def _tc_edge1(efp, yp, gp, bd1, b1t, bd2, b2t):
    return pl.pallas_call(
        _edge1_body,
        grid=(EGRID,),
        in_specs=[
            pl.BlockSpec((BE // 8, 64), lambda i: (i, 0)),
            pl.BlockSpec((BE // 8, 32), lambda i: (i, 0)),
            pl.BlockSpec((4, BE // 8, 128), lambda i: (0, i, 0)),
            _full((32, 128)),
            _full((32, 128)),
            _full((32, 128)),
            _full((32, 128)),
            _full((64, 512)),
            _full((1, 512)),
            _full((512, 512)),
            _full((1, 512)),
        ],
        out_specs=pl.BlockSpec((4, BE // 8, 128), lambda i: (0, i, 0)),
        out_shape=jax.ShapeDtypeStruct((4, E_PAD // 8, 128), _f32),
    )(efp, yp, gp, _QYX, _QYY, _QYZ, _QZ, bd1, b1t, bd2, b2t)


def _tc_prod(aggp, resp, ws, wv, with_v_res):
    if with_v_res:
        res_spec = pl.BlockSpec((4, BN // 8, 128), lambda i: (0, i, 0))
    else:
        res_spec = pl.BlockSpec((BN // 8, 128), lambda i: (i, 0))
    return pl.pallas_call(
        functools.partial(_prod_body, with_v_res),
        grid=(NGRID,),
        in_specs=[
            pl.BlockSpec((4, BN // 8, 128), lambda i: (0, i, 0)),
            res_spec,
            _full((5, EMB)),
            _full((4, EMB)),
        ],
        out_specs=pl.BlockSpec((4, BN // 8, 128), lambda i: (0, i, 0)),
        out_shape=jax.ShapeDtypeStruct((4, N_PAD // 8, 128), _f32),
    )(aggp, resp, ws, wv)


def _tc_pool(hallp, bidxp, lw, lb, ow, ob):
    nout = ow.shape[1]
    return pl.pallas_call(
        _pool_body,
        grid=(NGRID,),
        in_specs=[
            pl.BlockSpec((4, BN // 8, 128), lambda i: (0, i, 0)),
            pl.BlockSpec((BN // 8, 8), lambda i: (i, 0)),
            _full((64, 64)),
            _full((1, 64)),
            _full((64, nout)),
            _full((1, nout)),
        ],
        out_specs=pl.BlockSpec((N_GRAPHS, nout), lambda i: (0, 0)),
        out_shape=jax.ShapeDtypeStruct((N_GRAPHS, nout), _f32),
        scratch_shapes=[pltpu.VMEM((N_GRAPHS, 8 * EMB), _f32)],
    )(hallp, bidxp, lw, lb, ow, ob)


def kernel(x, coords_ca, bb_embs, edge_index, batch_idx,
           W_emb, b_emb, mlp0_W1, mlp0_b1, mlp0_W2, mlp0_b2,
           mlp1_W1, mlp1_b1, mlp1_W2, mlp1_b2,
           ws0, wv0, ws1, wv1, lin1_W, lin1_b, out_W, out_b):
    pad = E_PAD - N_EDGES
    zpad = jnp.zeros((pad,), jnp.int32)
    src_p = _perm_edges(jnp.concatenate([edge_index[0].astype(jnp.int32),
                                         zpad]))
    dst_p = _perm_edges(jnp.concatenate([edge_index[1].astype(jnp.int32),
                                         zpad]))
    cidx = jnp.concatenate([src_p, dst_p])
    dst2d = dst_p.reshape(-1, 128)
    coords16 = jnp.pad(coords_ca, ((0, 0), (0, 13)))
    npad = N_PAD - N_NODES
    xfp = jnp.concatenate(
        [x.astype(_f32), jnp.full((npad,), -1.0, _f32)]).reshape(N_PAD // 8, 8)
    bbp = jnp.pad(bb_embs, ((0, npad), (0, 0))).reshape(N_PAD // 8, 48)
    bidxp = jnp.concatenate(
        [batch_idx.astype(_f32),
         jnp.full((npad,), -1.0, _f32)]).reshape(N_PAD // 8, 8)

    hp = _tc_embed(xfp, bbp, W_emb, b_emb.reshape(1, -1))
    ccp = _sc_gather(coords16, cidx, 1024).reshape(-1, 128)
    hsp = _sc_gather(hp.reshape(N_PAD, EMB), src_p, 512).reshape(-1, 128)

    eye8 = jnp.eye(8, dtype=_f32)
    bd1_0 = jnp.einsum('ab,fk->afbk', eye8, mlp0_W1).reshape(64, 512)
    b1t0 = jnp.tile(mlp0_b1, 8).reshape(1, 512)
    bd2_0 = jnp.einsum('ab,kpj->akpbj', eye8,
                       mlp0_W2.reshape(64, 2, 16)).reshape(512, 256)
    b2t0 = jnp.tile(mlp0_b2.reshape(2, 1, 16), (1, 8, 1)).reshape(1, 256)
    bd1_1 = jnp.einsum('ab,fk->afbk', eye8, mlp1_W1).reshape(64, 512)
    b1t1 = jnp.tile(mlp1_b1, 8).reshape(1, 512)
    bd2_1 = jnp.einsum('ab,kpj->akpbj', eye8,
                       mlp1_W2.reshape(64, 4, 16)).reshape(512, 512)
    b2t1 = jnp.tile(mlp1_b2.reshape(4, 1, 16), (1, 8, 1)).reshape(1, 512)

    m0, efp, yp = _tc_edge0(ccp, hsp, bd1_0, b1t0, bd2_0, b2t0)
    agg0 = _sc_scatter_add(m0.reshape(4, E_PAD, EMB),
                           dst2d).reshape(4, N_PAD // 8, 128)
    hall1 = _tc_prod(agg0, hp, ws0, wv0, with_v_res=False)

    gp = _sc_gather4(hall1.reshape(4, N_PAD, EMB),
                     src_p, 512).reshape(4, E_PAD // 8, 128)
    m1 = _tc_edge1(efp, yp, gp, bd1_1, b1t1, bd2_1, b2t1)
    agg1 = _sc_scatter_add(m1.reshape(4, E_PAD, EMB),
                           dst2d).reshape(4, N_PAD // 8, 128)
    hall2 = _tc_prod(agg1, hall1, ws1, wv1, with_v_res=True)

    return _tc_pool(hall2, bidxp, lin1_W[_LIN1_PERM], lin1_b.reshape(1, -1),
                    out_W, out_b.reshape(1, -1))


# speedup vs baseline: 10.9120x; 1.0314x over previous
import functools

import numpy as np
import jax
import jax.numpy as jnp
from jax import lax
from jax.experimental import pallas as pl
from jax.experimental.pallas import tpu as pltpu
from jax.experimental.pallas import tpu_sc as plsc

N_NODES = 50000
N_PAD = 51200
N_EDGES = 800000
E_PAD = 802816
EMB = 16
R_MAX = 10.0
N_GRAPHS = 64

BE = 2048
EGRID = E_PAD // BE
BN = 2048
NGRID = N_PAD // BN

_f32 = jnp.float32
_bf16 = jnp.bfloat16

_SC_MESH = dict(core_axis_name="c", subcore_axis_name="s",
                num_cores=2, num_subcores=16)

_LIN1_PERM = np.concatenate([
    np.arange(16),
    np.array([16 + 3 * c + p for p in range(3) for c in range(16)]),
])


def _unpack(xp, n, w):
    return jnp.concatenate([xp[:, w * i:w * (i + 1)] for i in range(n)],
                           axis=0)


def _pack(xc, n):
    r = xc.shape[0] // n
    return jnp.concatenate([xc[r * i:r * (i + 1), :] for i in range(n)],
                           axis=1)


def _perm_edges(a):
    return a.reshape(-1, 8, BE // 8).transpose(0, 2, 1).reshape(-1)


def _np_sel(shape, entries):
    m = np.zeros(shape, np.float32)
    for r, c, v in entries:
        m[r, c] = v
    return m


_P3 = _np_sel((128, 8), [(16 * i + j, i, 1.0) for i in range(8) for j in range(3)])
_PX = _np_sel((128, 8), [(16 * i, i, 1.0) for i in range(8)])
_PY = _np_sel((128, 8), [(16 * i + 1, i, 1.0) for i in range(8)])
_PZ = _np_sel((128, 8), [(16 * i + 2, i, 1.0) for i in range(8)])
_Q8 = _np_sel((8, 64), [(i, 8 * i + f, 1.0) for i in range(8) for f in range(8)])
_Q8N = _np_sel((8, 64), [(i, 8 * i + f, (f + 1) * np.pi / R_MAX)
                         for i in range(8) for f in range(8)])
_Q16 = _np_sel((8, 128), [(i, 16 * i + j, 1.0)
                          for i in range(8) for j in range(16)])
_QYX = _np_sel((32, 128), [(i, 16 * i + j, 1.0)
                           for i in range(8) for j in range(16)])
_QYY = _np_sel((32, 128), [(8 + i, 16 * i + j, 1.0)
                           for i in range(8) for j in range(16)])
_QYZ = _np_sel((32, 128), [(16 + i, 16 * i + j, 1.0)
                           for i in range(8) for j in range(16)])
_QZ = _np_sel((32, 128), [(24 + i, 16 * i + j, 1.0)
                          for i in range(8) for j in range(16)])


def _sc_gather(table, idx, chunk):
    B = idx.shape[0]
    D = table.shape[1]
    per_w = B // 32
    iters = per_w // chunk
    nsub = chunk // 128
    mesh = plsc.VectorSubcoreMesh(**_SC_MESH)

    assert iters % 2 == 0

    @functools.partial(
        pl.kernel,
        out_type=jax.ShapeDtypeStruct((B, D), _f32),
        mesh=mesh,
        scratch_types=[
            pltpu.VMEM((2, chunk), jnp.int32),
            pltpu.VMEM((2, chunk, D), _f32),
            pltpu.SemaphoreType.DMA,
            pltpu.SemaphoreType.DMA,
        ],
        compiler_params=pltpu.CompilerParams(use_tc_tiling_on_sc=False),
    )
    def gk(table_hbm, idx_hbm, out_hbm, idx_v, rows_v, semg, semw):
        wid = lax.axis_index("s") * 2 + lax.axis_index("c")
        base = wid * per_w

        def load_idx(g, bf):
            pltpu.sync_copy(idx_hbm.at[pl.ds(base + g * chunk, chunk)],
                            idx_v.at[bf])

        def gathers(bf):
            return [
                pltpu.async_copy(
                    table_hbm.at[idx_v.at[bf].at[pl.ds(j * 128, 128)]],
                    rows_v.at[bf].at[pl.ds(j * 128, 128)], semg)
                for j in range(nsub)
            ]

        def wb(g, bf):
            return pltpu.async_copy(rows_v.at[bf],
                                    out_hbm.at[pl.ds(base + g * chunk, chunk)],
                                    semw)

        @pl.loop(0, iters, step=2)
        def _(g):
            load_idx(g, 0)
            d0 = gathers(0)
            load_idx(g + 1, 1)
            d1 = gathers(1)
            for d in d0:
                d.wait()
            w0 = wb(g, 0)
            for d in d1:
                d.wait()
            w1 = wb(g + 1, 1)
            w0.wait()
            w1.wait()

    return gk(table, idx)


def _sc_gather4(tables, idx, chunk):
    B = idx.shape[0]
    D = tables.shape[2]
    per_w = B // 32
    iters = per_w // chunk
    nsub = chunk // 128
    mesh = plsc.VectorSubcoreMesh(**_SC_MESH)

    assert iters % 2 == 0

    @functools.partial(
        pl.kernel,
        out_type=jax.ShapeDtypeStruct((4, B, D), _f32),
        mesh=mesh,
        scratch_types=[
            pltpu.VMEM((2, chunk), jnp.int32),
            pltpu.VMEM((2, 4, chunk, D), _f32),
            pltpu.SemaphoreType.DMA,
            pltpu.SemaphoreType.DMA,
        ],
        compiler_params=pltpu.CompilerParams(use_tc_tiling_on_sc=False),
    )
    def gk(tab_hbm, idx_hbm, out_hbm, idx_v, rows_v, semg, semw):
        wid = lax.axis_index("s") * 2 + lax.axis_index("c")
        base = wid * per_w

        def load_idx(g, bf):
            pltpu.sync_copy(idx_hbm.at[pl.ds(base + g * chunk, chunk)],
                            idx_v.at[bf])

        def gathers(bf):
            return [
                pltpu.async_copy(
                    tab_hbm.at[k].at[idx_v.at[bf].at[pl.ds(j * 128, 128)]],
                    rows_v.at[bf].at[k].at[pl.ds(j * 128, 128)], semg)
                for k in range(4)
                for j in range(nsub)
            ]

        def wbs(g, bf):
            return [
                pltpu.async_copy(rows_v.at[bf].at[k],
                                 out_hbm.at[k, pl.ds(base + g * chunk, chunk)],
                                 semw)
                for k in range(4)
            ]

        @pl.loop(0, iters, step=2)
        def _(g):
            load_idx(g, 0)
            d0 = gathers(0)
            load_idx(g + 1, 1)
            d1 = gathers(1)
            for d in d0:
                d.wait()
            w0 = wbs(g, 0)
            for d in d1:
                d.wait()
            w1 = wbs(g + 1, 1)
            for w in w0 + w1:
                w.wait()

    return gk(tables, idx)


def _sc_scatter_add(m, dst2d):
    rows_chunk = 512
    groups = E_PAD // rows_chunk
    per_sub = groups // 16
    rows_per_sub = N_PAD // 16
    zrows = 128
    mesh = plsc.VectorSubcoreMesh(**_SC_MESH)

    @functools.partial(
        pl.kernel,
        out_type=jax.ShapeDtypeStruct((4, N_PAD, EMB), _f32),
        mesh=mesh,
        scratch_types=[
            pltpu.VMEM((2, 4, 128), jnp.int32),
            pltpu.VMEM((2, rows_chunk, EMB), _f32),
            pltpu.VMEM((zrows, EMB), _f32),
            pltpu.VMEM_SHARED((N_PAD, EMB), _f32),
            pltpu.SemaphoreType.DMA,
            pltpu.SemaphoreType.DMA,
        ],
        compiler_params=pltpu.CompilerParams(use_tc_tiling_on_sc=False),
    )
    def sk(m_hbm, d_hbm, out_hbm, idx_v, m_v, zbuf, acc, semm, sema):
        c = lax.axis_index("c")
        sid = lax.axis_index("s")
        zero16 = jnp.zeros((16,), _f32)
        for r in range(zrows):
            zbuf[r, pl.ds(0, 16)] = zero16
        base_r = sid * rows_per_sub

        def adds(bf):
            return [
                pltpu.async_copy(m_v.at[bf].at[pl.ds(j * 128, 128)],
                                 acc.at[idx_v.at[bf].at[j]], sema, add=True)
                for j in range(4)
            ]

        for sub in range(2):
            grp_c = c * 2 + sub

            def load(g, bf, grp_c=grp_c):
                grp = sid * per_sub + g
                pltpu.sync_copy(
                    m_hbm.at[grp_c, pl.ds(grp * rows_chunk, rows_chunk)],
                    m_v.at[bf])
                pltpu.sync_copy(d_hbm.at[pl.ds(grp * 4, 4)], idx_v.at[bf])

            @pl.loop(0, rows_per_sub // zrows)
            def _(r):
                pltpu.sync_copy(zbuf, acc.at[pl.ds(base_r + r * zrows, zrows)])

            plsc.subcore_barrier()

            @pl.loop(0, per_sub, step=2)
            def _(g):
                load(g, 0)
                a0 = adds(0)
                load(g + 1, 1)
                a1 = adds(1)
                for a in a0 + a1:
                    a.wait()

            plsc.subcore_barrier()
            pltpu.sync_copy(acc.at[pl.ds(base_r, rows_per_sub)],
                            out_hbm.at[grp_c, pl.ds(base_r, rows_per_sub)])

    return sk(m, dst2d)


def _bdot(a, w, b):
    return lax.dot_general(a.astype(_bf16), w.astype(_bf16),
                           (((1,), (0,)), ((), ())),
                           preferred_element_type=_f32) + b[...]


def _hd(a, m):
    return lax.dot_general(a, m, (((1,), (0,)), ((), ())),
                           preferred_element_type=_f32,
                           precision=lax.Precision.HIGHEST)


def _embed_body(xfp_ref, bbp_ref, wemb_ref, bemb_ref, hp_ref):
    xf = _unpack(xfp_ref[...], 8, 1)
    bb = _unpack(bbp_ref[...], 8, 6)
    aa = lax.broadcasted_iota(jnp.int32, (BN, 26), 1).astype(_f32)
    onehot = (aa == xf).astype(_f32)
    feat = jnp.concatenate([onehot, bb], axis=1)
    hp_ref[...] = _pack(_bdot(feat, wemb_ref[...], bemb_ref), 8)


def _edge_mask(i):
    row = lax.broadcasted_iota(jnp.int32, (BE // 8, 128), 0)
    lane = lax.broadcasted_iota(jnp.int32, (BE // 8, 128), 1)
    eid = i * BE + (BE // 8) * lax.shift_right_logical(lane, 4) + row
    return eid < N_EDGES


def _emit_m(m_ref, k, val, valid, z16):
    nanv = jnp.full_like(val, jnp.nan)
    poisoned = jnp.where(z16 > 0.0, nanv, val)
    m_ref[k] = jnp.where(valid, poisoned, jnp.zeros_like(val))


def _edge0_body(csp_ref, cdp_ref, hsp_ref, p3_ref, px_ref, py_ref, pz_ref,
                q8_ref, q8n_ref, q16_ref, w1_ref, b1_ref, w2_ref, b2_ref,
                m_ref, ef_ref, y_ref):
    i = pl.program_id(0)
    d = csp_ref[...] - cdp_ref[...]
    len2 = _hd(d * d, p3_ref[...])
    l8 = jnp.sqrt(len2)
    u = l8 * (1.0 / R_MAX)
    u2 = u * u
    u5 = u2 * u2 * u
    env8 = jnp.where(u < 1.0,
                     1.0 - 21.0 * u5 + 35.0 * u5 * u - 15.0 * u5 * u2, 0.0)
    linv8 = 1.0 / (l8 + 1e-12)
    q8 = q8_ref[...]
    arg = _hd(l8, q8n_ref[...])
    l64 = _hd(l8, q8)
    bes = jnp.sqrt(2.0 / R_MAX) * jnp.sin(arg) / l64
    ef = bes * _hd(env8, q8)
    z8 = (l8 == 0.0).astype(_f32)
    z64 = _hd(z8, q8)
    ef_clean = jnp.where(z64 > 0.0, 0.0, ef)
    ef_ref[...] = ef_clean
    sq3 = jnp.sqrt(3.0)
    y1x8 = sq3 * _hd(d, px_ref[...]) * linv8
    y1y8 = sq3 * _hd(d, py_ref[...]) * linv8
    y1z8 = sq3 * _hd(d, pz_ref[...]) * linv8
    y_ref[...] = jnp.concatenate([y1x8, y1y8, y1z8, z8], axis=1)
    h1 = jnp.maximum(_bdot(ef_clean, w1_ref[...], b1_ref), 0.0)
    w_all = _bdot(h1, w2_ref[...], b2_ref)
    wp0 = w_all[:, 0:128]
    wp1 = w_all[:, 128:256]
    hs = hsp_ref[...]
    q16 = q16_ref[...]
    z16 = _hd(z8, q16)
    valid = _edge_mask(i)
    wh = wp1 * hs
    _emit_m(m_ref, 0, wp0 * hs, valid, z16)
    _emit_m(m_ref, 1, wh * _hd(y1x8, q16), valid, z16)
    _emit_m(m_ref, 2, wh * _hd(y1y8, q16), valid, z16)
    _emit_m(m_ref, 3, wh * _hd(y1z8, q16), valid, z16)


def _edge1_body(ef_ref, y_ref, gp_ref, qyx_ref, qyy_ref, qyz_ref, qz_ref,
                w1_ref, b1_ref, w2_ref, b2_ref, m_ref):
    i = pl.program_id(0)
    ef = ef_ref[...]
    y = y_ref[...]
    h1 = jnp.maximum(_bdot(ef, w1_ref[...], b1_ref), 0.0)
    w_all = _bdot(h1, w2_ref[...], b2_ref)
    wp0 = w_all[:, 0:128]
    wp1 = w_all[:, 128:256]
    wp2 = w_all[:, 256:384]
    wp3 = w_all[:, 384:512]
    y1x = _hd(y, qyx_ref[...])
    y1y = _hd(y, qyy_ref[...])
    y1z = _hd(y, qyz_ref[...])
    z16 = _hd(y, qz_ref[...])
    hs = gp_ref[0]
    hvx = gp_ref[1]
    hvy = gp_ref[2]
    hvz = gp_ref[3]
    dot = hvx * y1x + hvy * y1y + hvz * y1z
    valid = _edge_mask(i)
    wh = wp1 * hs
    _emit_m(m_ref, 0, wp0 * hs + wp3 * dot, valid, z16)
    _emit_m(m_ref, 1, wh * y1x + wp2 * hvx, valid, z16)
    _emit_m(m_ref, 2, wh * y1y + wp2 * hvy, valid, z16)
    _emit_m(m_ref, 3, wh * y1z + wp2 * hvz, valid, z16)


def _prod_body(with_v_res, aggp_ref, resp_ref, ws_ref, wv_ref, hallp_ref):
    s = _unpack(aggp_ref[0], 8, 16)
    vx = _unpack(aggp_ref[1], 8, 16)
    vy = _unpack(aggp_ref[2], 8, 16)
    vz = _unpack(aggp_ref[3], 8, 16)
    vv = vx * vx + vy * vy + vz * vz
    s2 = s * s
    ws = ws_ref[...]
    wv = wv_ref[...]
    out_s = (ws[0:1] * s + ws[1:2] * s2 + ws[2:3] * vv + ws[3:4] * (s2 * s)
             + ws[4:5] * (s * vv))
    coefv = wv[0:1] + wv[1:2] * s + wv[2:3] * vv + wv[3:4] * s2
    out_vx = coefv * vx
    out_vy = coefv * vy
    out_vz = coefv * vz
    if with_v_res:
        out_s = out_s + _unpack(resp_ref[0], 8, 16)
        out_vx = out_vx + _unpack(resp_ref[1], 8, 16)
        out_vy = out_vy + _unpack(resp_ref[2], 8, 16)
        out_vz = out_vz + _unpack(resp_ref[3], 8, 16)
    else:
        out_s = out_s + _unpack(resp_ref[...], 8, 16)
    hallp_ref[0] = _pack(out_s, 8)
    hallp_ref[1] = _pack(out_vx, 8)
    hallp_ref[2] = _pack(out_vy, 8)
    hallp_ref[3] = _pack(out_vz, 8)


def _pool_body(hallp_ref, bidxp_ref, lw_ref, lb_ref, ow_ref, ob_ref,
               out_ref, acc_ref):
    i = pl.program_id(0)

    @pl.when(i == 0)
    def _():
        acc_ref[...] = jnp.zeros((N_GRAPHS, 8 * EMB), _f32)

    h = jnp.concatenate([_unpack(hallp_ref[k], 8, 16) for k in range(4)],
                        axis=1)
    isn = jnp.sum(jnp.where(jnp.isnan(h), 1.0, 0.0), axis=1, keepdims=True)
    flag = jnp.where(isn > 0.0, 1.0, 0.0)
    h_clean = jnp.where(isn > 0.0, 0.0, h)
    h_ext = jnp.concatenate(
        [h_clean, jnp.broadcast_to(flag, (BN, 4 * EMB))], axis=1)
    bidx = _unpack(bidxp_ref[...], 8, 1)
    gids = lax.broadcasted_iota(jnp.int32, (BN, N_GRAPHS), 1).astype(_f32)
    onehot = (gids == bidx).astype(_f32)
    acc_ref[...] += lax.dot_general(onehot, h_ext, (((0,), (0,)), ((), ())),
                                    preferred_element_type=_f32,
                                    precision=lax.Precision.HIGHEST)

    @pl.when(i == NGRID - 1)
    def _():
        pooled = acc_ref[:, 0:4 * EMB]
        cnt = acc_ref[:, 4 * EMB:4 * EMB + 1]
        hmid = jnp.maximum(_bdot(pooled, lw_ref[...], lb_ref), 0.0)
        out = _bdot(hmid, ow_ref[...], ob_ref)
        out_ref[...] = jnp.where(cnt > 0.0, jnp.full_like(out, jnp.nan), out)


def _full(shape):
    return pl.BlockSpec(shape, lambda i: tuple(0 for _ in shape))


def _tc_embed(xfp, bbp, wemb, bemb):
    return pl.pallas_call(
        _embed_body,
        grid=(NGRID,),
        in_specs=[
            pl.BlockSpec((BN // 8, 8), lambda i: (i, 0)),
            pl.BlockSpec((BN // 8, 48), lambda i: (i, 0)),
            _full((32, EMB)),
            _full((1, EMB)),
        ],
        out_specs=pl.BlockSpec((BN // 8, 128), lambda i: (i, 0)),
        out_shape=jax.ShapeDtypeStruct((N_PAD // 8, 128), _f32),
    )(xfp, bbp, wemb, bemb)


def _tc_edge0(ccp, hsp, bd1, b1t, bd2, b2t):
    return pl.pallas_call(
        _edge0_body,
        grid=(EGRID,),
        in_specs=[
            pl.BlockSpec((BE // 8, 128), lambda i: (i, 0)),
            pl.BlockSpec((BE // 8, 128), lambda i: (i + EGRID, 0)),
            pl.BlockSpec((BE // 8, 128), lambda i: (i, 0)),
            _full((128, 8)),
            _full((128, 8)),
            _full((128, 8)),
            _full((128, 8)),
            _full((8, 64)),
            _full((8, 64)),
            _full((8, 128)),
            _full((64, 512)),
            _full((1, 512)),
            _full((512, 256)),
            _full((1, 256)),
        ],
        out_specs=[
            pl.BlockSpec((4, BE // 8, 128), lambda i: (0, i, 0)),
            pl.BlockSpec((BE // 8, 64), lambda i: (i, 0)),
            pl.BlockSpec((BE // 8, 32), lambda i: (i, 0)),
        ],
        out_shape=[
            jax.ShapeDtypeStruct((4, E_PAD // 8, 128), _f32),
            jax.ShapeDtypeStruct((E_PAD // 8, 64), _f32),
            jax.ShapeDtypeStruct((E_PAD // 8, 32), _f32),
        ],
    )(ccp, ccp, hsp, _P3, _PX, _PY, _PZ, _Q8, _Q8N, _Q16,
      bd1, b1t, bd2, b2t)


def _tc_edge1(efp, yp, gp, bd1, b1t, bd2, b2t):
    return pl.pallas_call(
        _edge1_body,
        grid=(EGRID,),
        in_specs=[
            pl.BlockSpec((BE // 8, 64), lambda i: (i, 0)),
            pl.BlockSpec((BE // 8, 32), lambda i: (i, 0)),
            pl.BlockSpec((4, BE // 8, 128), lambda i: (0, i, 0)),
            _full((32, 128)),
            _full((32, 128)),
            _full((32, 128)),
            _full((32, 128)),
            _full((64, 512)),
            _full((1, 512)),
            _full((512, 512)),
            _full((1, 512)),
        ],
        out_specs=pl.BlockSpec((4, BE // 8, 128), lambda i: (0, i, 0)),
        out_shape=jax.ShapeDtypeStruct((4, E_PAD // 8, 128), _f32),
    )(efp, yp, gp, _QYX, _QYY, _QYZ, _QZ, bd1, b1t, bd2, b2t)


def _tc_prod(aggp, resp, ws, wv, with_v_res):
    if with_v_res:
        res_spec = pl.BlockSpec((4, BN // 8, 128), lambda i: (0, i, 0))
    else:
        res_spec = pl.BlockSpec((BN // 8, 128), lambda i: (i, 0))
    return pl.pallas_call(
        functools.partial(_prod_body, with_v_res),
        grid=(NGRID,),
        in_specs=[
            pl.BlockSpec((4, BN // 8, 128), lambda i: (0, i, 0)),
            res_spec,
            _full((5, EMB)),
            _full((4, EMB)),
        ],
        out_specs=pl.BlockSpec((4, BN // 8, 128), lambda i: (0, i, 0)),
        out_shape=jax.ShapeDtypeStruct((4, N_PAD // 8, 128), _f32),
    )(aggp, resp, ws, wv)


def _tc_pool(hallp, bidxp, lw, lb, ow, ob):
    nout = ow.shape[1]
    return pl.pallas_call(
        _pool_body,
        grid=(NGRID,),
        in_specs=[
            pl.BlockSpec((4, BN // 8, 128), lambda i: (0, i, 0)),
            pl.BlockSpec((BN // 8, 8), lambda i: (i, 0)),
            _full((64, 64)),
            _full((1, 64)),
            _full((64, nout)),
            _full((1, nout)),
        ],
        out_specs=pl.BlockSpec((N_GRAPHS, nout), lambda i: (0, 0)),
        out_shape=jax.ShapeDtypeStruct((N_GRAPHS, nout), _f32),
        scratch_shapes=[pltpu.VMEM((N_GRAPHS, 8 * EMB), _f32)],
    )(hallp, bidxp, lw, lb, ow, ob)


def kernel(x, coords_ca, bb_embs, edge_index, batch_idx,
           W_emb, b_emb, mlp0_W1, mlp0_b1, mlp0_W2, mlp0_b2,
           mlp1_W1, mlp1_b1, mlp1_W2, mlp1_b2,
           ws0, wv0, ws1, wv1, lin1_W, lin1_b, out_W, out_b):
    pad = E_PAD - N_EDGES
    zpad = jnp.zeros((pad,), jnp.int32)
    src_p = _perm_edges(jnp.concatenate([edge_index[0].astype(jnp.int32),
                                         zpad]))
    dst_p = _perm_edges(jnp.concatenate([edge_index[1].astype(jnp.int32),
                                         zpad]))
    cidx = jnp.concatenate([src_p, dst_p])
    dst2d = dst_p.reshape(-1, 128)
    coords16 = jnp.pad(coords_ca, ((0, 0), (0, 13)))
    npad = N_PAD - N_NODES
    xfp = jnp.concatenate(
        [x.astype(_f32), jnp.full((npad,), -1.0, _f32)]).reshape(N_PAD // 8, 8)
    bbp = jnp.pad(bb_embs, ((0, npad), (0, 0))).reshape(N_PAD // 8, 48)
    bidxp = jnp.concatenate(
        [batch_idx.astype(_f32),
         jnp.full((npad,), -1.0, _f32)]).reshape(N_PAD // 8, 8)

    hp = _tc_embed(xfp, bbp, W_emb, b_emb.reshape(1, -1))
    ccp = _sc_gather(coords16, cidx, 1792).reshape(-1, 128)
    hsp = _sc_gather(hp.reshape(N_PAD, EMB), src_p, 1792).reshape(-1, 128)

    eye8 = jnp.eye(8, dtype=_f32)
    bd1_0 = jnp.einsum('ab,fk->afbk', eye8, mlp0_W1).reshape(64, 512)
    b1t0 = jnp.tile(mlp0_b1, 8).reshape(1, 512)
    bd2_0 = jnp.einsum('ab,kpj->akpbj', eye8,
                       mlp0_W2.reshape(64, 2, 16)).reshape(512, 256)
    b2t0 = jnp.tile(mlp0_b2.reshape(2, 1, 16), (1, 8, 1)).reshape(1, 256)
    bd1_1 = jnp.einsum('ab,fk->afbk', eye8, mlp1_W1).reshape(64, 512)
    b1t1 = jnp.tile(mlp1_b1, 8).reshape(1, 512)
    bd2_1 = jnp.einsum('ab,kpj->akpbj', eye8,
                       mlp1_W2.reshape(64, 4, 16)).reshape(512, 512)
    b2t1 = jnp.tile(mlp1_b2.reshape(4, 1, 16), (1, 8, 1)).reshape(1, 512)

    m0, efp, yp = _tc_edge0(ccp, hsp, bd1_0, b1t0, bd2_0, b2t0)
    agg0 = _sc_scatter_add(m0.reshape(4, E_PAD, EMB),
                           dst2d).reshape(4, N_PAD // 8, 128)
    hall1 = _tc_prod(agg0, hp, ws0, wv0, with_v_res=False)

    gp = _sc_gather4(hall1.reshape(4, N_PAD, EMB),
                     src_p, 896).reshape(4, E_PAD // 8, 128)
    m1 = _tc_edge1(efp, yp, gp, bd1_1, b1t1, bd2_1, b2t1)
    agg1 = _sc_scatter_add(m1.reshape(4, E_PAD, EMB),
                           dst2d).reshape(4, N_PAD // 8, 128)
    hall2 = _tc_prod(agg1, hall1, ws1, wv1, with_v_res=True)

    return _tc_pool(hall2, bidxp, lin1_W[_LIN1_PERM], lin1_b.reshape(1, -1),
                    out_W, out_b.reshape(1, -1))
